# Initial kernel scaffold; baseline (speedup 1.0000x reference)
import jax
import jax.numpy as jnp
from jax.experimental import pallas as pl


def kernel(edge_index, review_feat, score, ci, users, items, W_r1, W_r2, S1, S2, S3, feature2, feature3, P1_w, P1_b, P2_w, P2_b):
    raise NotImplementedError("write your pallas kernel here")


# speedup vs baseline: 7.8288x; 7.8288x over previous
import functools

import jax
import jax.numpy as jnp
from jax import lax
from jax.experimental import pallas as pl
from jax.experimental.pallas import tpu as pltpu
from jax.experimental.pallas import tpu_sc as plsc

N = 50000
NP_ = 51200
E = 800000
D = 64
H = 32
B = 16384
NS = 16
NC = 2
R = E // 128
RPT_BASE = R // NS
RPT_EXTRA = R % NS
CROWS = 4
NBIG = RPT_BASE // CROWS
NPT = NP_ // NS
BPT = B // NS

_SC_PARAMS = pltpu.CompilerParams(use_tc_tiling_on_sc=False)


def _tile_row_range(s):
    start = s * RPT_BASE + jnp.minimum(s, RPT_EXTRA)
    cnt = RPT_BASE + jnp.where(s < RPT_EXTRA, 1, 0)
    return start, cnt


def _sc_seg_sum_body(dst2d, rf, z32,
                     h0, h1,
                     acc, didx, dbuf):
    c = lax.axis_index("c")
    s = lax.axis_index("s")

    pltpu.sync_copy(z32.at[pl.ds(s * NPT, NPT)], acc.at[pl.ds(s * NPT, NPT)])
    plsc.subcore_barrier()

    start, cnt = _tile_row_range(s)
    col = c * H

    def do_rows(r, nrows):
        pltpu.sync_copy(dst2d.at[pl.ds(r, nrows)], didx.at[pl.ds(0, nrows)])
        pltpu.sync_copy(rf.at[pl.ds(r * 128, nrows * 128), pl.ds(col, H)],
                        dbuf.at[pl.ds(0, nrows * 128)])
        for j in range(nrows):
            pltpu.sync_copy(dbuf.at[pl.ds(j * 128, 128)],
                            acc.at[didx.at[j]], add=True)

    pl.loop(0, NBIG)(lambda i: do_rows(start + i * CROWS, CROWS))

    def rem(i):
        do_rows(start + NBIG * CROWS + i, 1)
    pl.loop(0, cnt - NBIG * CROWS)(rem)

    plsc.subcore_barrier()
    sl = pl.ds(s * NPT, NPT)

    @pl.when(c == 0)
    def _():
        pltpu.sync_copy(acc.at[sl], h0.at[sl])

    @pl.when(c == 1)
    def _():
        pltpu.sync_copy(acc.at[sl], h1.at[sl])


def _sc_seg_sum(dst2d, rf, z32):
    mesh = plsc.VectorSubcoreMesh(core_axis_name="c", subcore_axis_name="s")
    out = (
        jax.ShapeDtypeStruct((NP_, H), jnp.float32),
        jax.ShapeDtypeStruct((NP_, H), jnp.float32),
    )
    scratch = [
        pltpu.VMEM_SHARED((NP_, H), jnp.float32),
        pltpu.VMEM((CROWS, 128), jnp.int32),
        pltpu.VMEM((CROWS * 128, H), jnp.float32),
    ]
    return pl.kernel(_sc_seg_sum_body, out_type=out, mesh=mesh,
                     scratch_types=scratch,
                     compiler_params=_SC_PARAMS)(dst2d, rf, z32)


def _sc_deg_body(dst2d, z16, ones16, dg0, dg1, dacc, didx, ones_v):
    c = lax.axis_index("c")
    s = lax.axis_index("s")

    pltpu.sync_copy(z16.at[pl.ds(s * NPT, NPT)], dacc.at[pl.ds(s * NPT, NPT)])
    pltpu.sync_copy(ones16, ones_v)
    plsc.subcore_barrier()

    start, cnt = _tile_row_range(s)

    def do_rows(r, nrows):
        pltpu.sync_copy(dst2d.at[pl.ds(r, nrows)], didx.at[pl.ds(0, nrows)])
        for j in range(nrows):
            pltpu.sync_copy(ones_v, dacc.at[didx.at[j]], add=True)

    pl.loop(c, NBIG, step=2)(lambda i: do_rows(start + i * CROWS, CROWS))

    @pl.when(c == 0)
    def _():
        pl.loop(0, cnt - NBIG * CROWS)(
            lambda i: do_rows(start + NBIG * CROWS + i, 1))

    plsc.subcore_barrier()
    sl = pl.ds(s * NPT, NPT)

    @pl.when(c == 0)
    def _():
        pltpu.sync_copy(dacc.at[sl], dg0.at[sl])

    @pl.when(c == 1)
    def _():
        pltpu.sync_copy(dacc.at[sl], dg1.at[sl])


def _sc_deg(dst2d, z16, ones16):
    mesh = plsc.VectorSubcoreMesh(core_axis_name="c", subcore_axis_name="s")
    out = (
        jax.ShapeDtypeStruct((NP_, 16), jnp.float32),
        jax.ShapeDtypeStruct((NP_, 16), jnp.float32),
    )
    scratch = [
        pltpu.VMEM_SHARED((NP_, 16), jnp.float32),
        pltpu.VMEM((CROWS, 128), jnp.int32),
        pltpu.VMEM((128, 16), jnp.float32),
    ]
    return pl.kernel(_sc_deg_body, out_type=out, mesh=mesh,
                     scratch_types=scratch,
                     compiler_params=_SC_PARAMS)(dst2d, z16, ones16)


def _tc_tables_body(h0, h1, dg0, dg1, f2, civ, w2a, w2b, s2,
                    g5a, g5b, ci16):
    deg = jnp.maximum(dg0[...][:, :1] + dg1[...][:, :1], 1.0)
    inv = 1.0 / deg
    hr0 = h0[...] * inv
    hr1 = h1[...] * inv
    rf = (jnp.dot(hr0, w2a[...], preferred_element_type=jnp.float32)
          + jnp.dot(hr1, w2b[...], preferred_element_type=jnp.float32))
    g = (f2[...] + rf) * civ[...]
    g0 = g[:, :H]
    g1 = g[:, H:]
    s2m = s2[...]
    for k in range(5):
        g5a[k] = g0 * s2m[k, :H]
        g5b[k] = g1 * s2m[k, H:]
    ci16[...] = jnp.broadcast_to(civ[...], civ.shape[:1] + (16,))


def _tc_tables(h0, h1, dg0, dg1, feature2, ci, w2a, w2b, s2):
    bn = 1024
    grid = (NP_ // bn,)
    return pl.pallas_call(
        _tc_tables_body,
        grid=grid,
        in_specs=[
            pl.BlockSpec((bn, H), lambda i: (i, 0)),
            pl.BlockSpec((bn, H), lambda i: (i, 0)),
            pl.BlockSpec((bn, 16), lambda i: (i, 0)),
            pl.BlockSpec((bn, 16), lambda i: (i, 0)),
            pl.BlockSpec((bn, D), lambda i: (i, 0)),
            pl.BlockSpec((bn, 1), lambda i: (i, 0)),
            pl.BlockSpec((H, D), lambda i: (0, 0)),
            pl.BlockSpec((H, D), lambda i: (0, 0)),
            pl.BlockSpec((5, D), lambda i: (0, 0)),
        ],
        out_specs=[
            pl.BlockSpec((5, bn, H), lambda i: (0, i, 0)),
            pl.BlockSpec((5, bn, H), lambda i: (0, i, 0)),
            pl.BlockSpec((bn, 16), lambda i: (i, 0)),
        ],
        out_shape=[
            jax.ShapeDtypeStruct((5, NP_, H), jnp.float32),
            jax.ShapeDtypeStruct((5, NP_, H), jnp.float32),
            jax.ShapeDtypeStruct((NP_, 16), jnp.float32),
        ],
    )(h0, h1, dg0, dg1, feature2, ci, w2a, w2b, s2)


def _sc_edge_body(src2d, scr2d, dst2d, g5a, g5b, ci16t, users2d, items2d, z32,
                  u0, u1, i0, i1, ciu, cii,
                  acc, iv_src, iv_scr, iv_dst, gidx, gbuf, bv, bbuf, cibuf):
    c = lax.axis_index("c")
    s = lax.axis_index("s")

    pltpu.sync_copy(z32.at[pl.ds(s * NPT, NPT)], acc.at[pl.ds(s * NPT, NPT)])
    plsc.subcore_barrier()

    start, cnt = _tile_row_range(s)

    def do_rows(r, nrows):
        pltpu.sync_copy(src2d.at[pl.ds(r, nrows)], iv_src.at[pl.ds(0, nrows)])
        pltpu.sync_copy(scr2d.at[pl.ds(r, nrows)], iv_scr.at[pl.ds(0, nrows)])
        pltpu.sync_copy(dst2d.at[pl.ds(r, nrows)], iv_dst.at[pl.ds(0, nrows)])
        for j in range(nrows):
            for k in range(8):
                sl = pl.ds(k * 16, 16)
                gidx[j, sl] = iv_scr[j, sl] * NP_ + iv_src[j, sl]

        def gath(tab):
            for j in range(nrows):
                pltpu.sync_copy(tab.at[gidx.at[j]],
                                gbuf.at[pl.ds(j * 128, 128)])

        @pl.when(c == 0)
        def _():
            gath(g5a)

        @pl.when(c == 1)
        def _():
            gath(g5b)

        for j in range(nrows):
            pltpu.sync_copy(gbuf.at[pl.ds(j * 128, 128)],
                            acc.at[iv_dst.at[j]], add=True)

    pl.loop(0, NBIG)(lambda i: do_rows(start + i * CROWS, CROWS))
    pl.loop(0, cnt - NBIG * CROWS)(lambda i: do_rows(start + NBIG * CROWS + i, 1))

    plsc.subcore_barrier()

    def batch_gather(idx2d, out_half, ci_out):
        pltpu.sync_copy(idx2d.at[pl.ds(s * 8, 8)], bv)
        for j in range(8):
            dst_sl = pl.ds(s * BPT + j * 128, 128)
            pltpu.sync_copy(acc.at[bv.at[j]], bbuf)
            pltpu.sync_copy(bbuf, out_half.at[dst_sl])
            if ci_out is not None:
                pltpu.sync_copy(ci16t.at[bv.at[j]], cibuf)
                pltpu.sync_copy(cibuf, ci_out.at[dst_sl])

    @pl.when(c == 0)
    def _():
        batch_gather(users2d, u0, ciu)
        batch_gather(items2d, i0, None)

    @pl.when(c == 1)
    def _():
        batch_gather(users2d, u1, None)
        batch_gather(items2d, i1, cii)


def _sc_edge(src2d, scr2d, dst2d, g5a, g5b, ci16t, users2d, items2d, z32):
    mesh = plsc.VectorSubcoreMesh(core_axis_name="c", subcore_axis_name="s")
    out = (
        jax.ShapeDtypeStruct((B, H), jnp.float32),
        jax.ShapeDtypeStruct((B, H), jnp.float32),
        jax.ShapeDtypeStruct((B, H), jnp.float32),
        jax.ShapeDtypeStruct((B, H), jnp.float32),
        jax.ShapeDtypeStruct((B, 16), jnp.float32),
        jax.ShapeDtypeStruct((B, 16), jnp.float32),
    )
    scratch = [
        pltpu.VMEM_SHARED((NP_, H), jnp.float32),
        pltpu.VMEM((CROWS, 128), jnp.int32),
        pltpu.VMEM((CROWS, 128), jnp.int32),
        pltpu.VMEM((CROWS, 128), jnp.int32),
        pltpu.VMEM((CROWS, 128), jnp.int32),
        pltpu.VMEM((CROWS * 128, H), jnp.float32),
        pltpu.VMEM((8, 128), jnp.int32),
        pltpu.VMEM((128, H), jnp.float32),
        pltpu.VMEM((128, 16), jnp.float32),
    ]
    return pl.kernel(_sc_edge_body, out_type=out, mesh=mesh,
                     scratch_types=scratch, compiler_params=_SC_PARAMS)(
        src2d, scr2d, dst2d, g5a, g5b, ci16t, users2d, items2d, z32)


def _tc_head_body(u0, u1, i0, i1, ciu, cii, p1a, p1b, b1, p2t, b2, out):
    cc = ciu[:, 0:1] * cii[:, 0:1]
    x0 = u0[...] * i0[...] * cc
    x1 = u1[...] * i1[...] * cc
    h = (jnp.dot(x0, p1a[...], preferred_element_type=jnp.float32)
         + jnp.dot(x1, p1b[...], preferred_element_type=jnp.float32)
         + b1[...])
    h = jnp.where(h > 0, h, 0.1 * h)
    out[...] = jnp.dot(h, p2t[...], preferred_element_type=jnp.float32) + b2[...]


def _tc_head(u0, u1, i0, i1, ciu, cii, p1a, p1b, b1, p2t, b2):
    bb = 1024
    grid = (B // bb,)
    return pl.pallas_call(
        _tc_head_body,
        grid=grid,
        in_specs=[
            pl.BlockSpec((bb, H), lambda i: (i, 0)),
            pl.BlockSpec((bb, H), lambda i: (i, 0)),
            pl.BlockSpec((bb, H), lambda i: (i, 0)),
            pl.BlockSpec((bb, H), lambda i: (i, 0)),
            pl.BlockSpec((bb, 16), lambda i: (i, 0)),
            pl.BlockSpec((bb, 16), lambda i: (i, 0)),
            pl.BlockSpec((H, D), lambda i: (0, 0)),
            pl.BlockSpec((H, D), lambda i: (0, 0)),
            pl.BlockSpec((1, D), lambda i: (0, 0)),
            pl.BlockSpec((D, 5), lambda i: (0, 0)),
            pl.BlockSpec((1, 5), lambda i: (0, 0)),
        ],
        out_specs=pl.BlockSpec((bb, 5), lambda i: (i, 0)),
        out_shape=jax.ShapeDtypeStruct((B, 5), jnp.float32),
    )(u0, u1, i0, i1, ciu, cii, p1a, p1b, b1, p2t, b2)


def kernel(edge_index, review_feat, score, ci, users, items,
           W_r1, W_r2, S1, S2, S3, feature2, feature3,
           P1_w, P1_b, P2_w, P2_b):
    src2d = edge_index[0].astype(jnp.int32).reshape(R, 128)
    dst2d = edge_index[1].astype(jnp.int32).reshape(R, 128)
    scr2d = score.astype(jnp.int32).reshape(R, 128)
    users2d = users.astype(jnp.int32).reshape(128, 128)
    items2d = items.astype(jnp.int32).reshape(128, 128)

    z32 = jnp.zeros((NP_, H), jnp.float32)
    z16 = jnp.zeros((NP_, 16), jnp.float32)
    ones16 = jnp.ones((128, 16), jnp.float32)
    f2p = jnp.pad(feature2, ((0, NP_ - N), (0, 0)))
    cip = jnp.pad(ci, ((0, NP_ - N), (0, 0)))

    h0, h1 = _sc_seg_sum(dst2d, review_feat, z32)
    dg0, dg1 = _sc_deg(dst2d, z16, ones16)

    w2t = W_r2.T
    g5a, g5b, ci16t = _tc_tables(h0, h1, dg0, dg1, f2p, cip,
                                 w2t[:H], w2t[H:], S2)

    u0, u1, i0, i1, ciu, cii = _sc_edge(
        src2d, scr2d, dst2d,
        g5a.reshape(5 * NP_, H), g5b.reshape(5 * NP_, H),
        ci16t, users2d, items2d, z32)

    p1t = P1_w.T
    return _tc_head(u0, u1, i0, i1, ciu, cii,
                    p1t[:H], p1t[H:], P1_b.reshape(1, D),
                    P2_w.T, P2_b.reshape(1, 5))


# speedup vs baseline: 9.4883x; 1.2120x over previous
import jax
import jax.numpy as jnp
from jax import lax
from jax.experimental import pallas as pl
from jax.experimental.pallas import tpu as pltpu
from jax.experimental.pallas import tpu_sc as plsc

N = 50000
NP_ = 51200
E = 800000
D = 64
H = 32
B = 16384
NS = 16
CH = 256
NCH = E // CH
CPT_BASE = NCH // NS
CPT_EXTRA = NCH % NS
NPT = NP_ // NS
BPT = B // NS

_SC_PARAMS = pltpu.CompilerParams(use_tc_tiling_on_sc=False)


def _tile_chunk_range(s):
    start = s * CPT_BASE + jnp.minimum(s, CPT_EXTRA)
    cnt = CPT_BASE + jnp.where(s < CPT_EXTRA, 1, 0)
    return start, cnt


def _fill_rows(buf, rows, cols, value):
    v = jnp.full((16,), value, jnp.float32)
    for r in range(rows):
        for k in range(cols // 16):
            buf[r, pl.ds(k * 16, 16)] = v


def _zero_shared_slice(zbuf, zrows, shared, s):
    nrep = NPT // zrows
    rem = NPT - nrep * zrows

    def rep(i):
        pltpu.sync_copy(zbuf, shared.at[pl.ds(s * NPT + i * zrows, zrows)])

    pl.loop(0, nrep)(rep)
    if rem:
        pltpu.sync_copy(zbuf.at[pl.ds(0, rem)],
                        shared.at[pl.ds(s * NPT + nrep * zrows, rem)])


def _sc_seg_sum_body(dst1d, rf, h0, h1,
                     acc, dv0, dv1, db0, db1, sem0, sem1):
    c = lax.axis_index("c")
    s = lax.axis_index("s")
    col = c * H

    _fill_rows(db0, CH, H, 0.0)
    _zero_shared_slice(db0, CH, acc, s)
    plsc.subcore_barrier()

    start, cnt = _tile_chunk_range(s)

    def pair(p):
        t0 = start + 2 * p
        t1 = t0 + 1
        l0 = pltpu.async_copy(rf.at[pl.ds(t0 * CH, CH), pl.ds(col, H)],
                              db0, sem0)
        l1 = pltpu.async_copy(rf.at[pl.ds(t1 * CH, CH), pl.ds(col, H)],
                              db1, sem1)
        pltpu.sync_copy(dst1d.at[pl.ds(t0 * CH, CH)], dv0)
        pltpu.sync_copy(dst1d.at[pl.ds(t1 * CH, CH)], dv1)
        l0.wait()
        s0 = pltpu.async_copy(db0, acc.at[dv0], sem0, add=True)
        l1.wait()
        s1 = pltpu.async_copy(db1, acc.at[dv1], sem1, add=True)
        s0.wait()
        s1.wait()

    pl.loop(0, cnt // 2)(pair)

    @pl.when(cnt % 2 == 1)
    def _():
        t = start + cnt - 1
        pltpu.sync_copy(dst1d.at[pl.ds(t * CH, CH)], dv0)
        pltpu.sync_copy(rf.at[pl.ds(t * CH, CH), pl.ds(col, H)], db0)
        pltpu.sync_copy(db0, acc.at[dv0], add=True)

    plsc.subcore_barrier()
    sl = pl.ds(s * NPT, NPT)

    @pl.when(c == 0)
    def _():
        pltpu.sync_copy(acc.at[sl], h0.at[sl])

    @pl.when(c == 1)
    def _():
        pltpu.sync_copy(acc.at[sl], h1.at[sl])


def _sc_seg_sum(dst1d, rf):
    mesh = plsc.VectorSubcoreMesh(core_axis_name="c", subcore_axis_name="s")
    out = (
        jax.ShapeDtypeStruct((NP_, H), jnp.float32),
        jax.ShapeDtypeStruct((NP_, H), jnp.float32),
    )
    scratch = [
        pltpu.VMEM_SHARED((NP_, H), jnp.float32),
        pltpu.VMEM((CH,), jnp.int32),
        pltpu.VMEM((CH,), jnp.int32),
        pltpu.VMEM((CH, H), jnp.float32),
        pltpu.VMEM((CH, H), jnp.float32),
        pltpu.SemaphoreType.DMA,
        pltpu.SemaphoreType.DMA,
    ]
    return pl.kernel(_sc_seg_sum_body, out_type=out, mesh=mesh,
                     scratch_types=scratch,
                     compiler_params=_SC_PARAMS)(dst1d, rf)


def _sc_deg_body2(dst1d, dg0, dg1, dacc, dv0, dv1, ones_v, zb, sem0, sem1):
    c = lax.axis_index("c")
    s = lax.axis_index("s")

    _fill_rows(ones_v, CH, 16, 1.0)
    _fill_rows(zb, CH, 16, 0.0)
    _zero_shared_slice(zb, CH, dacc, s)
    plsc.subcore_barrier()

    start, cnt = _tile_chunk_range(s)
    npair = (cnt - c + 3) // 4

    def pair(p):
        i0 = c + 4 * p
        t0 = start + i0
        pltpu.sync_copy(dst1d.at[pl.ds(t0 * CH, CH)], dv0)
        s0 = pltpu.async_copy(ones_v, dacc.at[dv0], sem0, add=True)

        @pl.when(i0 + 2 < cnt)
        def _():
            t1 = start + i0 + 2
            pltpu.sync_copy(dst1d.at[pl.ds(t1 * CH, CH)], dv1)
            s1 = pltpu.async_copy(ones_v, dacc.at[dv1], sem1, add=True)
            s1.wait()

        s0.wait()

    pl.loop(0, npair)(pair)

    plsc.subcore_barrier()
    sl = pl.ds(s * NPT, NPT)

    @pl.when(c == 0)
    def _():
        pltpu.sync_copy(dacc.at[sl], dg0.at[sl])

    @pl.when(c == 1)
    def _():
        pltpu.sync_copy(dacc.at[sl], dg1.at[sl])


def _sc_deg(dst1d):
    mesh = plsc.VectorSubcoreMesh(core_axis_name="c", subcore_axis_name="s")
    out = (
        jax.ShapeDtypeStruct((NP_, 16), jnp.float32),
        jax.ShapeDtypeStruct((NP_, 16), jnp.float32),
    )
    scratch = [
        pltpu.VMEM_SHARED((NP_, 16), jnp.float32),
        pltpu.VMEM((CH,), jnp.int32),
        pltpu.VMEM((CH,), jnp.int32),
        pltpu.VMEM((CH, 16), jnp.float32),
        pltpu.VMEM((CH, 16), jnp.float32),
        pltpu.SemaphoreType.DMA,
        pltpu.SemaphoreType.DMA,
    ]
    return pl.kernel(_sc_deg_body2, out_type=out, mesh=mesh,
                     scratch_types=scratch,
                     compiler_params=_SC_PARAMS)(dst1d)


def _tc_tables_body(h0, h1, dg0, dg1, f2, civ, w2a, w2b, s2,
                    g5a, g5b, ci16):
    deg = jnp.maximum(dg0[...][:, :1] + dg1[...][:, :1], 1.0)
    inv = 1.0 / deg
    hr0 = h0[...] * inv
    hr1 = h1[...] * inv
    rf = (jnp.dot(hr0, w2a[...], preferred_element_type=jnp.float32)
          + jnp.dot(hr1, w2b[...], preferred_element_type=jnp.float32))
    g = (f2[...] + rf) * civ[...]
    g0 = g[:, :H]
    g1 = g[:, H:]
    s2m = s2[...]
    for k in range(5):
        g5a[k] = g0 * s2m[k, :H]
        g5b[k] = g1 * s2m[k, H:]
    ci16[...] = jnp.broadcast_to(civ[...], civ.shape[:1] + (16,))


def _tc_tables(h0, h1, dg0, dg1, feature2, ci, w2a, w2b, s2):
    bn = 1024
    grid = (NP_ // bn,)
    return pl.pallas_call(
        _tc_tables_body,
        grid=grid,
        in_specs=[
            pl.BlockSpec((bn, H), lambda i: (i, 0)),
            pl.BlockSpec((bn, H), lambda i: (i, 0)),
            pl.BlockSpec((bn, 16), lambda i: (i, 0)),
            pl.BlockSpec((bn, 16), lambda i: (i, 0)),
            pl.BlockSpec((bn, D), lambda i: (i, 0)),
            pl.BlockSpec((bn, 1), lambda i: (i, 0)),
            pl.BlockSpec((H, D), lambda i: (0, 0)),
            pl.BlockSpec((H, D), lambda i: (0, 0)),
            pl.BlockSpec((5, D), lambda i: (0, 0)),
        ],
        out_specs=[
            pl.BlockSpec((5, bn, H), lambda i: (0, i, 0)),
            pl.BlockSpec((5, bn, H), lambda i: (0, i, 0)),
            pl.BlockSpec((bn, 16), lambda i: (i, 0)),
        ],
        out_shape=[
            jax.ShapeDtypeStruct((5, NP_, H), jnp.float32),
            jax.ShapeDtypeStruct((5, NP_, H), jnp.float32),
            jax.ShapeDtypeStruct((NP_, 16), jnp.float32),
        ],
    )(h0, h1, dg0, dg1, feature2, ci, w2a, w2b, s2)


def _sc_edge_body(ss1d, dst1d, g5a, g5b, ci16t, users1d, items1d,
                  emb, cic,
                  acc, sv0, sv1, dv0, dv1, gx0, gx1, gb0, gb1, uv, cb,
                  sem0, sem1):
    c = lax.axis_index("c")
    s = lax.axis_index("s")

    _fill_rows(gb0, CH, H, 0.0)
    _zero_shared_slice(gb0, CH, acc, s)
    plsc.subcore_barrier()

    start, cnt = _tile_chunk_range(s)

    def load_idx(t, sv, dv):
        pltpu.sync_copy(ss1d.at[pl.ds(t * 2 * CH, 2 * CH)], sv)
        pltpu.sync_copy(dst1d.at[pl.ds(t * CH, CH)], dv)

    def build_gidx(sv, gx):
        for k in range(CH // 16):
            sl = pl.ds(k * 16, 16)
            gx[sl] = sv[pl.ds(CH + k * 16, 16)] * NP_ + sv[sl]

    def pair(p, tab):
        t0 = start + 2 * p
        t1 = t0 + 1
        load_idx(t0, sv0, dv0)
        build_gidx(sv0, gx0)
        g0 = pltpu.async_copy(tab.at[gx0], gb0, sem0)
        load_idx(t1, sv1, dv1)
        build_gidx(sv1, gx1)
        g1 = pltpu.async_copy(tab.at[gx1], gb1, sem1)
        g0.wait()
        s0 = pltpu.async_copy(gb0, acc.at[dv0], sem0, add=True)
        g1.wait()
        s1 = pltpu.async_copy(gb1, acc.at[dv1], sem1, add=True)
        s0.wait()
        s1.wait()

    def tail(tab):
        @pl.when(cnt % 2 == 1)
        def _():
            t = start + cnt - 1
            load_idx(t, sv0, dv0)
            build_gidx(sv0, gx0)
            pltpu.sync_copy(tab.at[gx0], gb0)
            pltpu.sync_copy(gb0, acc.at[dv0], add=True)

    @pl.when(c == 0)
    def _():
        pl.loop(0, cnt // 2)(lambda p: pair(p, g5a))
        tail(g5a)

    @pl.when(c == 1)
    def _():
        pl.loop(0, cnt // 2)(lambda p: pair(p, g5b))
        tail(g5b)

    plsc.subcore_barrier()

    def bgather(idx1d, col_off, ci_col):
        pltpu.sync_copy(idx1d.at[pl.ds(s * BPT, BPT)], uv)
        ws = []
        for q in range(4):
            gb, sem = (gb0, sem0) if q % 2 == 0 else (gb1, sem1)
            if q >= 2:
                ws[q - 2].wait()
            pltpu.async_copy(acc.at[uv.at[pl.ds(q * CH, CH)]], gb, sem).wait()
            w = pltpu.async_copy(
                gb, emb.at[pl.ds(s * BPT + q * CH, CH), pl.ds(col_off, H)],
                sem)
            ws.append(w)
        ws[2].wait()
        ws[3].wait()
        if ci_col is not None:
            for q in range(4):
                pltpu.sync_copy(ci16t.at[uv.at[pl.ds(q * CH, CH)]], cb)
                pltpu.sync_copy(
                    cb, cic.at[pl.ds(s * BPT + q * CH, CH), pl.ds(ci_col, 16)])

    @pl.when(c == 0)
    def _():
        bgather(users1d, 0, 0)
        bgather(items1d, 2 * H, None)

    @pl.when(c == 1)
    def _():
        bgather(users1d, H, None)
        bgather(items1d, 3 * H, 16)


def _sc_edge(ss1d, dst1d, g5a, g5b, ci16t, users1d, items1d):
    mesh = plsc.VectorSubcoreMesh(core_axis_name="c", subcore_axis_name="s")
    out = (
        jax.ShapeDtypeStruct((B, 4 * H), jnp.float32),
        jax.ShapeDtypeStruct((B, 2 * 16), jnp.float32),
    )
    scratch = [
        pltpu.VMEM_SHARED((NP_, H), jnp.float32),
        pltpu.VMEM((2 * CH,), jnp.int32),
        pltpu.VMEM((2 * CH,), jnp.int32),
        pltpu.VMEM((CH,), jnp.int32),
        pltpu.VMEM((CH,), jnp.int32),
        pltpu.VMEM((CH,), jnp.int32),
        pltpu.VMEM((CH,), jnp.int32),
        pltpu.VMEM((CH, H), jnp.float32),
        pltpu.VMEM((CH, H), jnp.float32),
        pltpu.VMEM((BPT,), jnp.int32),
        pltpu.VMEM((CH, 16), jnp.float32),
        pltpu.SemaphoreType.DMA,
        pltpu.SemaphoreType.DMA,
    ]
    return pl.kernel(_sc_edge_body, out_type=out, mesh=mesh,
                     scratch_types=scratch, compiler_params=_SC_PARAMS)(
        ss1d, dst1d, g5a, g5b, ci16t, users1d, items1d)


def _tc_head_body(emb, cic, p1a, p1b, b1, p2t, b2, out):
    e = emb[...]
    cc = cic[:, 0:1] * cic[:, 16:17]
    x0 = e[:, 0:H] * e[:, 2 * H:3 * H] * cc
    x1 = e[:, H:2 * H] * e[:, 3 * H:4 * H] * cc
    h = (jnp.dot(x0, p1a[...], preferred_element_type=jnp.float32)
         + jnp.dot(x1, p1b[...], preferred_element_type=jnp.float32)
         + b1[...])
    h = jnp.where(h > 0, h, 0.1 * h)
    out[...] = jnp.dot(h, p2t[...], preferred_element_type=jnp.float32) + b2[...]


def _tc_head(emb, cic, p1a, p1b, b1, p2t, b2):
    bb = 2048
    grid = (B // bb,)
    return pl.pallas_call(
        _tc_head_body,
        grid=grid,
        in_specs=[
            pl.BlockSpec((bb, 4 * H), lambda i: (i, 0)),
            pl.BlockSpec((bb, 32), lambda i: (i, 0)),
            pl.BlockSpec((H, D), lambda i: (0, 0)),
            pl.BlockSpec((H, D), lambda i: (0, 0)),
            pl.BlockSpec((1, D), lambda i: (0, 0)),
            pl.BlockSpec((D, 5), lambda i: (0, 0)),
            pl.BlockSpec((1, 5), lambda i: (0, 0)),
        ],
        out_specs=pl.BlockSpec((bb, 5), lambda i: (i, 0)),
        out_shape=jax.ShapeDtypeStruct((B, 5), jnp.float32),
    )(emb, cic, p1a, p1b, b1, p2t, b2)


def kernel(edge_index, review_feat, score, ci, users, items,
           W_r1, W_r2, S1, S2, S3, feature2, feature3,
           P1_w, P1_b, P2_w, P2_b):
    src = edge_index[0].astype(jnp.int32)
    dst1d = edge_index[1].astype(jnp.int32)
    scr = score.astype(jnp.int32)
    ss1d = jnp.concatenate(
        [src.reshape(NCH, CH), scr.reshape(NCH, CH)], axis=1).reshape(-1)
    users1d = users.astype(jnp.int32)
    items1d = items.astype(jnp.int32)

    f2p = jnp.pad(feature2, ((0, NP_ - N), (0, 0)))
    cip = jnp.pad(ci, ((0, NP_ - N), (0, 0)))

    h0, h1 = _sc_seg_sum(dst1d, review_feat)
    dg0, dg1 = _sc_deg(dst1d)

    w2t = W_r2.T
    g5a, g5b, ci16t = _tc_tables(h0, h1, dg0, dg1, f2p, cip,
                                 w2t[:H], w2t[H:], S2)

    emb, cic = _sc_edge(
        ss1d, dst1d,
        g5a.reshape(5 * NP_, H), g5b.reshape(5 * NP_, H),
        ci16t, users1d, items1d)

    p1t = P1_w.T
    return _tc_head(emb, cic, p1t[:H], p1t[H:], P1_b.reshape(1, D),
                    P2_w.T, P2_b.reshape(1, 5))


# speedup vs baseline: 11.8829x; 1.2524x over previous
import jax
import jax.numpy as jnp
from jax import lax
from jax.experimental import pallas as pl
from jax.experimental.pallas import tpu as pltpu
from jax.experimental.pallas import tpu_sc as plsc

N = 50000
NP_ = 51200
E = 800000
D = 64
H = 32
B = 16384
NS = 16
CH = 256
NCH = E // CH
CPT_BASE = NCH // NS
CPT_EXTRA = NCH % NS
NPT = NP_ // NS
BPT = B // NS

_SC_PARAMS = pltpu.CompilerParams(use_tc_tiling_on_sc=False)


def _tile_chunk_range(s):
    start = s * CPT_BASE + jnp.minimum(s, CPT_EXTRA)
    cnt = CPT_BASE + jnp.where(s < CPT_EXTRA, 1, 0)
    return start, cnt


def _fill_rows(buf, rows, cols, value):
    v = jnp.full((16,), value, jnp.float32)
    for r in range(rows):
        for k in range(cols // 16):
            buf[r, pl.ds(k * 16, 16)] = v


def _zero_shared_slice(zbuf, zrows, shared, s):
    nrep = NPT // zrows
    rem = NPT - nrep * zrows

    def rep(i):
        pltpu.sync_copy(zbuf, shared.at[pl.ds(s * NPT + i * zrows, zrows)])

    pl.loop(0, nrep)(rep)
    if rem:
        pltpu.sync_copy(zbuf.at[pl.ds(0, rem)],
                        shared.at[pl.ds(s * NPT + nrep * zrows, rem)])


def _sc_seg_sum_body(dst1d, rf, h0, h1,
                     acc, dv0, dv1, db0, db1, sem0, sem1):
    c = lax.axis_index("c")
    s = lax.axis_index("s")
    col = c * H

    _fill_rows(db0, CH, H, 0.0)
    _zero_shared_slice(db0, CH, acc, s)
    plsc.subcore_barrier()

    start, cnt = _tile_chunk_range(s)

    def pair(p):
        t0 = start + 2 * p
        t1 = t0 + 1
        l0 = pltpu.async_copy(rf.at[pl.ds(t0 * CH, CH), pl.ds(col, H)],
                              db0, sem0)
        l1 = pltpu.async_copy(rf.at[pl.ds(t1 * CH, CH), pl.ds(col, H)],
                              db1, sem1)
        pltpu.sync_copy(dst1d.at[pl.ds(t0 * CH, CH)], dv0)
        pltpu.sync_copy(dst1d.at[pl.ds(t1 * CH, CH)], dv1)
        l0.wait()
        s0 = pltpu.async_copy(db0, acc.at[dv0], sem0, add=True)
        l1.wait()
        s1 = pltpu.async_copy(db1, acc.at[dv1], sem1, add=True)
        s0.wait()
        s1.wait()

    pl.loop(0, cnt // 2)(pair)

    @pl.when(cnt % 2 == 1)
    def _():
        t = start + cnt - 1
        pltpu.sync_copy(dst1d.at[pl.ds(t * CH, CH)], dv0)
        pltpu.sync_copy(rf.at[pl.ds(t * CH, CH), pl.ds(col, H)], db0)
        pltpu.sync_copy(db0, acc.at[dv0], add=True)

    plsc.subcore_barrier()
    sl = pl.ds(s * NPT, NPT)

    @pl.when(c == 0)
    def _():
        pltpu.sync_copy(acc.at[sl], h0.at[sl])

    @pl.when(c == 1)
    def _():
        pltpu.sync_copy(acc.at[sl], h1.at[sl])


def _sc_seg_sum(dst1d, rf):
    mesh = plsc.VectorSubcoreMesh(core_axis_name="c", subcore_axis_name="s")
    out = (
        jax.ShapeDtypeStruct((NP_, H), jnp.float32),
        jax.ShapeDtypeStruct((NP_, H), jnp.float32),
    )
    scratch = [
        pltpu.VMEM_SHARED((NP_, H), jnp.float32),
        pltpu.VMEM((CH,), jnp.int32),
        pltpu.VMEM((CH,), jnp.int32),
        pltpu.VMEM((CH, H), jnp.float32),
        pltpu.VMEM((CH, H), jnp.float32),
        pltpu.SemaphoreType.DMA,
        pltpu.SemaphoreType.DMA,
    ]
    return pl.kernel(_sc_seg_sum_body, out_type=out, mesh=mesh,
                     scratch_types=scratch,
                     compiler_params=_SC_PARAMS)(dst1d, rf)


def _sc_deg_body2(dst1d, dg0, dg1, dacc, dv0, dv1, ones_v, zb, sem0, sem1):
    c = lax.axis_index("c")
    s = lax.axis_index("s")

    _fill_rows(ones_v, CH, 16, 1.0)
    _fill_rows(zb, CH, 16, 0.0)
    _zero_shared_slice(zb, CH, dacc, s)
    plsc.subcore_barrier()

    start, cnt = _tile_chunk_range(s)
    npair = (cnt - c + 3) // 4

    def pair(p):
        i0 = c + 4 * p
        t0 = start + i0
        pltpu.sync_copy(dst1d.at[pl.ds(t0 * CH, CH)], dv0)
        s0 = pltpu.async_copy(ones_v, dacc.at[dv0], sem0, add=True)

        @pl.when(i0 + 2 < cnt)
        def _():
            t1 = start + i0 + 2
            pltpu.sync_copy(dst1d.at[pl.ds(t1 * CH, CH)], dv1)
            s1 = pltpu.async_copy(ones_v, dacc.at[dv1], sem1, add=True)
            s1.wait()

        s0.wait()

    pl.loop(0, npair)(pair)

    plsc.subcore_barrier()
    sl = pl.ds(s * NPT, NPT)

    @pl.when(c == 0)
    def _():
        pltpu.sync_copy(dacc.at[sl], dg0.at[sl])

    @pl.when(c == 1)
    def _():
        pltpu.sync_copy(dacc.at[sl], dg1.at[sl])


def _sc_deg(dst1d):
    mesh = plsc.VectorSubcoreMesh(core_axis_name="c", subcore_axis_name="s")
    out = (
        jax.ShapeDtypeStruct((NP_, 16), jnp.float32),
        jax.ShapeDtypeStruct((NP_, 16), jnp.float32),
    )
    scratch = [
        pltpu.VMEM_SHARED((NP_, 16), jnp.float32),
        pltpu.VMEM((CH,), jnp.int32),
        pltpu.VMEM((CH,), jnp.int32),
        pltpu.VMEM((CH, 16), jnp.float32),
        pltpu.VMEM((CH, 16), jnp.float32),
        pltpu.SemaphoreType.DMA,
        pltpu.SemaphoreType.DMA,
    ]
    return pl.kernel(_sc_deg_body2, out_type=out, mesh=mesh,
                     scratch_types=scratch,
                     compiler_params=_SC_PARAMS)(dst1d)


def _tc_tables_body(h0p, h1p, f2ap, f2bp, invp, cip,
                    m0a, m1a, m0b, m1b, s2ta, s2tb,
                    g5a, g5b):
    rfa = (jnp.dot(h0p[...], m0a[...], preferred_element_type=jnp.float32)
           + jnp.dot(h1p[...], m1a[...], preferred_element_type=jnp.float32))
    rfb = (jnp.dot(h0p[...], m0b[...], preferred_element_type=jnp.float32)
           + jnp.dot(h1p[...], m1b[...], preferred_element_type=jnp.float32))
    iv = invp[...]
    cv = cip[...]
    ga = (f2ap[...] + rfa * iv) * cv
    gb = (f2bp[...] + rfb * iv) * cv
    for k in range(5):
        g5a[k] = ga * s2ta[k, :]
        g5b[k] = gb * s2tb[k, :]


def _tc_tables(h0p, h1p, f2ap, f2bp, invp, cip,
               m0a, m1a, m0b, m1b, s2ta, s2tb):
    bn4 = 256
    grid = (NP_ // (4 * bn4),)
    full = lambda shape: pl.BlockSpec(shape, lambda i: tuple(0 for _ in shape))
    row = pl.BlockSpec((bn4, 128), lambda i: (i, 0))
    return pl.pallas_call(
        _tc_tables_body,
        grid=grid,
        in_specs=[row, row, row, row, row, row,
                  full((128, 128)), full((128, 128)),
                  full((128, 128)), full((128, 128)),
                  full((5, 128)), full((5, 128))],
        out_specs=[
            pl.BlockSpec((5, bn4, 128), lambda i: (0, i, 0)),
            pl.BlockSpec((5, bn4, 128), lambda i: (0, i, 0)),
        ],
        out_shape=[
            jax.ShapeDtypeStruct((5, NP_ // 4, 128), jnp.float32),
            jax.ShapeDtypeStruct((5, NP_ // 4, 128), jnp.float32),
        ],
    )(h0p, h1p, f2ap, f2bp, invp, cip, m0a, m1a, m0b, m1b, s2ta, s2tb)


def _sc_edge_body(ss1d, dst1d, g5a, g5b, ci16t, users1d, items1d,
                  emb, cic,
                  acc, sv0, sv1, dv0, dv1, gx0, gx1, gb0, gb1, uv, cb,
                  sem0, sem1):
    c = lax.axis_index("c")
    s = lax.axis_index("s")

    _fill_rows(gb0, CH, H, 0.0)
    _zero_shared_slice(gb0, CH, acc, s)
    plsc.subcore_barrier()

    start, cnt = _tile_chunk_range(s)

    def load_idx(t, sv, dv):
        pltpu.sync_copy(ss1d.at[pl.ds(t * 2 * CH, 2 * CH)], sv)
        pltpu.sync_copy(dst1d.at[pl.ds(t * CH, CH)], dv)

    def build_gidx(sv, gx):
        for k in range(CH // 16):
            sl = pl.ds(k * 16, 16)
            gx[sl] = sv[pl.ds(CH + k * 16, 16)] * NP_ + sv[sl]

    def pair(p, tab):
        t0 = start + 2 * p
        t1 = t0 + 1
        load_idx(t0, sv0, dv0)
        build_gidx(sv0, gx0)
        g0 = pltpu.async_copy(tab.at[gx0], gb0, sem0)
        load_idx(t1, sv1, dv1)
        build_gidx(sv1, gx1)
        g1 = pltpu.async_copy(tab.at[gx1], gb1, sem1)
        g0.wait()
        s0 = pltpu.async_copy(gb0, acc.at[dv0], sem0, add=True)
        g1.wait()
        s1 = pltpu.async_copy(gb1, acc.at[dv1], sem1, add=True)
        s0.wait()
        s1.wait()

    def tail(tab):
        @pl.when(cnt % 2 == 1)
        def _():
            t = start + cnt - 1
            load_idx(t, sv0, dv0)
            build_gidx(sv0, gx0)
            pltpu.sync_copy(tab.at[gx0], gb0)
            pltpu.sync_copy(gb0, acc.at[dv0], add=True)

    @pl.when(c == 0)
    def _():
        pl.loop(0, cnt // 2)(lambda p: pair(p, g5a))
        tail(g5a)

    @pl.when(c == 1)
    def _():
        pl.loop(0, cnt // 2)(lambda p: pair(p, g5b))
        tail(g5b)

    plsc.subcore_barrier()

    def bgather(idx1d, col_off, ci_col):
        pltpu.sync_copy(idx1d.at[pl.ds(s * BPT, BPT)], uv)
        ws = []
        for q in range(4):
            gb, sem = (gb0, sem0) if q % 2 == 0 else (gb1, sem1)
            if q >= 2:
                ws[q - 2].wait()
            pltpu.async_copy(acc.at[uv.at[pl.ds(q * CH, CH)]], gb, sem).wait()
            w = pltpu.async_copy(
                gb, emb.at[pl.ds(s * BPT + q * CH, CH), pl.ds(col_off, H)],
                sem)
            ws.append(w)
        ws[2].wait()
        ws[3].wait()
        if ci_col is not None:
            for q in range(4):
                pltpu.sync_copy(ci16t.at[uv.at[pl.ds(q * CH, CH)]], cb)
                pltpu.sync_copy(
                    cb, cic.at[pl.ds(s * BPT + q * CH, CH), pl.ds(ci_col, 16)])

    @pl.when(c == 0)
    def _():
        bgather(users1d, 0, 0)
        bgather(items1d, 2 * H, None)

    @pl.when(c == 1)
    def _():
        bgather(users1d, H, None)
        bgather(items1d, 3 * H, 16)


def _sc_edge(ss1d, dst1d, g5a, g5b, ci16t, users1d, items1d):
    mesh = plsc.VectorSubcoreMesh(core_axis_name="c", subcore_axis_name="s")
    out = (
        jax.ShapeDtypeStruct((B, 4 * H), jnp.float32),
        jax.ShapeDtypeStruct((B, 2 * 16), jnp.float32),
    )
    scratch = [
        pltpu.VMEM_SHARED((NP_, H), jnp.float32),
        pltpu.VMEM((2 * CH,), jnp.int32),
        pltpu.VMEM((2 * CH,), jnp.int32),
        pltpu.VMEM((CH,), jnp.int32),
        pltpu.VMEM((CH,), jnp.int32),
        pltpu.VMEM((CH,), jnp.int32),
        pltpu.VMEM((CH,), jnp.int32),
        pltpu.VMEM((CH, H), jnp.float32),
        pltpu.VMEM((CH, H), jnp.float32),
        pltpu.VMEM((BPT,), jnp.int32),
        pltpu.VMEM((CH, 16), jnp.float32),
        pltpu.SemaphoreType.DMA,
        pltpu.SemaphoreType.DMA,
    ]
    return pl.kernel(_sc_edge_body, out_type=out, mesh=mesh,
                     scratch_types=scratch, compiler_params=_SC_PARAMS)(
        ss1d, dst1d, g5a, g5b, ci16t, users1d, items1d)


def _tc_head_body(emb, cic, p1a, p1b, b1, p2t, b2, out):
    e = emb[...]
    cc = cic[:, 0:1] * cic[:, 16:17]
    x0 = e[:, 0:H] * e[:, 2 * H:3 * H] * cc
    x1 = e[:, H:2 * H] * e[:, 3 * H:4 * H] * cc
    h = (jnp.dot(x0, p1a[...], preferred_element_type=jnp.float32)
         + jnp.dot(x1, p1b[...], preferred_element_type=jnp.float32)
         + b1[...])
    h = jnp.where(h > 0, h, 0.1 * h)
    out[...] = jnp.dot(h, p2t[...], preferred_element_type=jnp.float32) + b2[...]


def _tc_head(emb, cic, p1a, p1b, b1, p2t, b2):
    bb = 2048
    grid = (B // bb,)
    return pl.pallas_call(
        _tc_head_body,
        grid=grid,
        in_specs=[
            pl.BlockSpec((bb, 4 * H), lambda i: (i, 0)),
            pl.BlockSpec((bb, 32), lambda i: (i, 0)),
            pl.BlockSpec((H, D), lambda i: (0, 0)),
            pl.BlockSpec((H, D), lambda i: (0, 0)),
            pl.BlockSpec((1, D), lambda i: (0, 0)),
            pl.BlockSpec((D, 5), lambda i: (0, 0)),
            pl.BlockSpec((1, 5), lambda i: (0, 0)),
        ],
        out_specs=pl.BlockSpec((bb, 5), lambda i: (i, 0)),
        out_shape=jax.ShapeDtypeStruct((B, 5), jnp.float32),
    )(emb, cic, p1a, p1b, b1, p2t, b2)


def kernel(edge_index, review_feat, score, ci, users, items,
           W_r1, W_r2, S1, S2, S3, feature2, feature3,
           P1_w, P1_b, P2_w, P2_b):
    src = edge_index[0].astype(jnp.int32)
    dst1d = edge_index[1].astype(jnp.int32)
    scr = score.astype(jnp.int32)
    ss1d = jnp.concatenate(
        [src.reshape(NCH, CH), scr.reshape(NCH, CH)], axis=1).reshape(-1)
    users1d = users.astype(jnp.int32)
    items1d = items.astype(jnp.int32)

    f2p = jnp.pad(feature2, ((0, NP_ - N), (0, 0)))
    cip = jnp.pad(ci, ((0, NP_ - N), (0, 0)))

    h0, h1 = _sc_seg_sum(dst1d, review_feat)
    dg0, dg1 = _sc_deg(dst1d)

    h0p = h0.reshape(NP_ // 4, 128)
    h1p = h1.reshape(NP_ // 4, 128)
    deg16 = (dg0.reshape(NP_ // 8, 128)
             + dg1.reshape(NP_ // 8, 128)).reshape(NP_, 16)
    inv = 1.0 / jnp.maximum(deg16[:, :1], 1.0)
    invp = jnp.broadcast_to(inv, (NP_, H)).reshape(NP_ // 4, 128)
    cip32 = jnp.broadcast_to(cip, (NP_, H)).reshape(NP_ // 4, 128)
    f2ap = f2p[:, :H].reshape(NP_ // 4, 128)
    f2bp = f2p[:, H:].reshape(NP_ // 4, 128)
    ci16t = jnp.broadcast_to(cip, (NP_, 16)).reshape(NP_ // 8, 128)

    w2t = W_r2.T
    eye4 = jnp.eye(4, dtype=jnp.float32)
    m0a = jnp.kron(eye4, w2t[:H, :H])
    m1a = jnp.kron(eye4, w2t[H:, :H])
    m0b = jnp.kron(eye4, w2t[:H, H:])
    m1b = jnp.kron(eye4, w2t[H:, H:])
    s2ta = jnp.tile(S2[:, :H], (1, 4))
    s2tb = jnp.tile(S2[:, H:], (1, 4))

    g5a_p, g5b_p = _tc_tables(h0p, h1p, f2ap, f2bp, invp, cip32,
                              m0a, m1a, m0b, m1b, s2ta, s2tb)

    emb, cic = _sc_edge(
        ss1d, dst1d,
        g5a_p.reshape(5 * NP_, H), g5b_p.reshape(5 * NP_, H),
        ci16t.reshape(NP_, 16), users1d, items1d)

    p1t = P1_w.T
    return _tc_head(emb, cic, p1t[:H], p1t[H:], P1_b.reshape(1, D),
                    P2_w.T, P2_b.reshape(1, 5))


# speedup vs baseline: 13.0385x; 1.0973x over previous
import jax
import jax.numpy as jnp
from jax import lax
from jax.experimental import pallas as pl
from jax.experimental.pallas import tpu as pltpu
from jax.experimental.pallas import tpu_sc as plsc

N = 50000
NP_ = 51200
E = 800000
D = 64
H = 32
B = 16384
NS = 16
CH = 256
NCH = E // CH
CPT_BASE = NCH // NS
CPT_EXTRA = NCH % NS
NPT = NP_ // NS
BPT = B // NS

_SC_PARAMS = pltpu.CompilerParams(use_tc_tiling_on_sc=False)


def _tile_chunk_range(s):
    start = s * CPT_BASE + jnp.minimum(s, CPT_EXTRA)
    cnt = CPT_BASE + jnp.where(s < CPT_EXTRA, 1, 0)
    return start, cnt


def _fill_rows(buf, rows, cols, value):
    v = jnp.full((16,), value, jnp.float32)
    for r in range(rows):
        for k in range(cols // 16):
            buf[r, pl.ds(k * 16, 16)] = v


def _zero_shared_slice(zbuf, zrows, shared, s):
    nrep = NPT // zrows
    rem = NPT - nrep * zrows

    def rep(i):
        pltpu.sync_copy(zbuf, shared.at[pl.ds(s * NPT + i * zrows, zrows)])

    pl.loop(0, nrep)(rep)
    if rem:
        pltpu.sync_copy(zbuf.at[pl.ds(0, rem)],
                        shared.at[pl.ds(s * NPT + nrep * zrows, rem)])


def _sc_seg_sum_body(dst1d, rf, h0, h1,
                     acc, dv0, dv1, db0, db1, sem0, sem1, semi):
    c = lax.axis_index("c")
    s = lax.axis_index("s")
    col = c * H

    _fill_rows(db0, CH, H, 0.0)
    _zero_shared_slice(db0, CH, acc, s)
    plsc.subcore_barrier()

    start, cnt = _tile_chunk_range(s)

    def pair(p):
        t0 = start + 2 * p
        t1 = t0 + 1
        l0 = pltpu.async_copy(rf.at[pl.ds(t0 * CH, CH), pl.ds(col, H)],
                              db0, sem0)
        l1 = pltpu.async_copy(rf.at[pl.ds(t1 * CH, CH), pl.ds(col, H)],
                              db1, sem1)
        i0 = pltpu.async_copy(dst1d.at[pl.ds(t0 * CH, CH)], dv0, semi)
        i1 = pltpu.async_copy(dst1d.at[pl.ds(t1 * CH, CH)], dv1, semi)
        l0.wait()
        i0.wait()
        s0 = pltpu.async_copy(db0, acc.at[dv0], sem0, add=True)
        l1.wait()
        i1.wait()
        s1 = pltpu.async_copy(db1, acc.at[dv1], sem1, add=True)
        s0.wait()
        s1.wait()

    pl.loop(0, cnt // 2)(pair)

    @pl.when(cnt % 2 == 1)
    def _():
        t = start + cnt - 1
        pltpu.sync_copy(dst1d.at[pl.ds(t * CH, CH)], dv0)
        pltpu.sync_copy(rf.at[pl.ds(t * CH, CH), pl.ds(col, H)], db0)
        pltpu.sync_copy(db0, acc.at[dv0], add=True)

    plsc.subcore_barrier()
    sl = pl.ds(s * NPT, NPT)

    @pl.when(c == 0)
    def _():
        pltpu.sync_copy(acc.at[sl], h0.at[sl])

    @pl.when(c == 1)
    def _():
        pltpu.sync_copy(acc.at[sl], h1.at[sl])


def _sc_seg_sum(dst1d, rf):
    mesh = plsc.VectorSubcoreMesh(core_axis_name="c", subcore_axis_name="s")
    out = (
        jax.ShapeDtypeStruct((NP_, H), jnp.float32),
        jax.ShapeDtypeStruct((NP_, H), jnp.float32),
    )
    scratch = [
        pltpu.VMEM_SHARED((NP_, H), jnp.float32),
        pltpu.VMEM((CH,), jnp.int32),
        pltpu.VMEM((CH,), jnp.int32),
        pltpu.VMEM((CH, H), jnp.float32),
        pltpu.VMEM((CH, H), jnp.float32),
        pltpu.SemaphoreType.DMA,
        pltpu.SemaphoreType.DMA,
        pltpu.SemaphoreType.DMA,
    ]
    return pl.kernel(_sc_seg_sum_body, out_type=out, mesh=mesh,
                     scratch_types=scratch,
                     compiler_params=_SC_PARAMS)(dst1d, rf)


def _sc_deg_body2(dst1d, dg0, dg1, dacc, dv0, dv1, ones_v, zb, sem0, sem1):
    c = lax.axis_index("c")
    s = lax.axis_index("s")

    _fill_rows(ones_v, CH, 16, 1.0)
    _fill_rows(zb, CH, 16, 0.0)
    _zero_shared_slice(zb, CH, dacc, s)
    plsc.subcore_barrier()

    start, cnt = _tile_chunk_range(s)
    npair = (cnt - c + 3) // 4

    def pair(p):
        i0 = c + 4 * p
        t0 = start + i0
        pltpu.sync_copy(dst1d.at[pl.ds(t0 * CH, CH)], dv0)
        s0 = pltpu.async_copy(ones_v, dacc.at[dv0], sem0, add=True)

        @pl.when(i0 + 2 < cnt)
        def _():
            t1 = start + i0 + 2
            pltpu.sync_copy(dst1d.at[pl.ds(t1 * CH, CH)], dv1)
            s1 = pltpu.async_copy(ones_v, dacc.at[dv1], sem1, add=True)
            s1.wait()

        s0.wait()

    pl.loop(0, npair)(pair)

    plsc.subcore_barrier()
    sl = pl.ds(s * NPT, NPT)

    @pl.when(c == 0)
    def _():
        pltpu.sync_copy(dacc.at[sl], dg0.at[sl])

    @pl.when(c == 1)
    def _():
        pltpu.sync_copy(dacc.at[sl], dg1.at[sl])


def _sc_deg(dst1d):
    mesh = plsc.VectorSubcoreMesh(core_axis_name="c", subcore_axis_name="s")
    out = (
        jax.ShapeDtypeStruct((NP_, 16), jnp.float32),
        jax.ShapeDtypeStruct((NP_, 16), jnp.float32),
    )
    scratch = [
        pltpu.VMEM_SHARED((NP_, 16), jnp.float32),
        pltpu.VMEM((CH,), jnp.int32),
        pltpu.VMEM((CH,), jnp.int32),
        pltpu.VMEM((CH, 16), jnp.float32),
        pltpu.VMEM((CH, 16), jnp.float32),
        pltpu.SemaphoreType.DMA,
        pltpu.SemaphoreType.DMA,
    ]
    return pl.kernel(_sc_deg_body2, out_type=out, mesh=mesh,
                     scratch_types=scratch,
                     compiler_params=_SC_PARAMS)(dst1d)


def _tc_tables_body(h0p, h1p, f2ap, f2bp, invp, cip,
                    m0a, m1a, m0b, m1b, s2ta, s2tb,
                    g5a, g5b):
    rfa = (jnp.dot(h0p[...], m0a[...], preferred_element_type=jnp.float32)
           + jnp.dot(h1p[...], m1a[...], preferred_element_type=jnp.float32))
    rfb = (jnp.dot(h0p[...], m0b[...], preferred_element_type=jnp.float32)
           + jnp.dot(h1p[...], m1b[...], preferred_element_type=jnp.float32))
    iv = invp[...]
    cv = cip[...]
    ga = (f2ap[...] + rfa * iv) * cv
    gb = (f2bp[...] + rfb * iv) * cv
    for k in range(5):
        g5a[k] = ga * s2ta[k, :]
        g5b[k] = gb * s2tb[k, :]


def _tc_tables(h0p, h1p, f2ap, f2bp, invp, cip,
               m0a, m1a, m0b, m1b, s2ta, s2tb):
    bn4 = 256
    grid = (NP_ // (4 * bn4),)
    full = lambda shape: pl.BlockSpec(shape, lambda i: tuple(0 for _ in shape))
    row = pl.BlockSpec((bn4, 128), lambda i: (i, 0))
    return pl.pallas_call(
        _tc_tables_body,
        grid=grid,
        in_specs=[row, row, row, row, row, row,
                  full((128, 128)), full((128, 128)),
                  full((128, 128)), full((128, 128)),
                  full((5, 128)), full((5, 128))],
        out_specs=[
            pl.BlockSpec((5, bn4, 128), lambda i: (0, i, 0)),
            pl.BlockSpec((5, bn4, 128), lambda i: (0, i, 0)),
        ],
        out_shape=[
            jax.ShapeDtypeStruct((5, NP_ // 4, 128), jnp.float32),
            jax.ShapeDtypeStruct((5, NP_ // 4, 128), jnp.float32),
        ],
    )(h0p, h1p, f2ap, f2bp, invp, cip, m0a, m1a, m0b, m1b, s2ta, s2tb)


def _sc_edge_body(ss1d, dst1d, g5a, g5b, ci16t, users1d, items1d,
                  emb, cic,
                  acc, sv0, sv1, dv0, dv1, gx0, gx1, gb0, gb1, uv, cb,
                  sem0, sem1, semi):
    c = lax.axis_index("c")
    s = lax.axis_index("s")

    _fill_rows(gb0, CH, H, 0.0)
    _zero_shared_slice(gb0, CH, acc, s)
    plsc.subcore_barrier()

    start, cnt = _tile_chunk_range(s)

    def build_gidx(sv, gx):
        for k in range(CH // 16):
            sl = pl.ds(k * 16, 16)
            gx[sl] = sv[pl.ds(CH + k * 16, 16)] * NP_ + sv[sl]

    def pair(p, tab):
        t0 = start + 2 * p
        t1 = t0 + 1
        a0 = pltpu.async_copy(ss1d.at[pl.ds(t0 * 2 * CH, 2 * CH)], sv0, semi)
        a1 = pltpu.async_copy(ss1d.at[pl.ds(t1 * 2 * CH, 2 * CH)], sv1, semi)
        b0 = pltpu.async_copy(dst1d.at[pl.ds(t0 * CH, CH)], dv0, semi)
        b1 = pltpu.async_copy(dst1d.at[pl.ds(t1 * CH, CH)], dv1, semi)
        a0.wait()
        build_gidx(sv0, gx0)
        g0 = pltpu.async_copy(tab.at[gx0], gb0, sem0)
        a1.wait()
        build_gidx(sv1, gx1)
        g1 = pltpu.async_copy(tab.at[gx1], gb1, sem1)
        g0.wait()
        b0.wait()
        s0 = pltpu.async_copy(gb0, acc.at[dv0], sem0, add=True)
        g1.wait()
        b1.wait()
        s1 = pltpu.async_copy(gb1, acc.at[dv1], sem1, add=True)
        s0.wait()
        s1.wait()

    def tail(tab):
        @pl.when(cnt % 2 == 1)
        def _():
            t = start + cnt - 1
            pltpu.sync_copy(ss1d.at[pl.ds(t * 2 * CH, 2 * CH)], sv0)
            pltpu.sync_copy(dst1d.at[pl.ds(t * CH, CH)], dv0)
            build_gidx(sv0, gx0)
            pltpu.sync_copy(tab.at[gx0], gb0)
            pltpu.sync_copy(gb0, acc.at[dv0], add=True)

    @pl.when(c == 0)
    def _():
        pl.loop(0, cnt // 2)(lambda p: pair(p, g5a))
        tail(g5a)

    @pl.when(c == 1)
    def _():
        pl.loop(0, cnt // 2)(lambda p: pair(p, g5b))
        tail(g5b)

    plsc.subcore_barrier()

    def bgather(idx1d, col_off, ci_col):
        pltpu.sync_copy(idx1d.at[pl.ds(s * BPT, BPT)], uv)
        ws = []
        for q in range(4):
            gb, sem = (gb0, sem0) if q % 2 == 0 else (gb1, sem1)
            if q >= 2:
                ws[q - 2].wait()
            pltpu.async_copy(acc.at[uv.at[pl.ds(q * CH, CH)]], gb, sem).wait()
            w = pltpu.async_copy(
                gb, emb.at[pl.ds(s * BPT + q * CH, CH), pl.ds(col_off, H)],
                sem)
            ws.append(w)
        ws[2].wait()
        ws[3].wait()
        if ci_col is not None:
            for q in range(4):
                pltpu.sync_copy(ci16t.at[uv.at[pl.ds(q * CH, CH)]], cb)
                pltpu.sync_copy(
                    cb, cic.at[pl.ds(s * BPT + q * CH, CH), pl.ds(ci_col, 16)])

    @pl.when(c == 0)
    def _():
        bgather(users1d, 0, 0)
        bgather(items1d, 2 * H, None)

    @pl.when(c == 1)
    def _():
        bgather(users1d, H, None)
        bgather(items1d, 3 * H, 16)


def _sc_edge(ss1d, dst1d, g5a, g5b, ci16t, users1d, items1d):
    mesh = plsc.VectorSubcoreMesh(core_axis_name="c", subcore_axis_name="s")
    out = (
        jax.ShapeDtypeStruct((B, 4 * H), jnp.float32),
        jax.ShapeDtypeStruct((B, 2 * 16), jnp.float32),
    )
    scratch = [
        pltpu.VMEM_SHARED((NP_, H), jnp.float32),
        pltpu.VMEM((2 * CH,), jnp.int32),
        pltpu.VMEM((2 * CH,), jnp.int32),
        pltpu.VMEM((CH,), jnp.int32),
        pltpu.VMEM((CH,), jnp.int32),
        pltpu.VMEM((CH,), jnp.int32),
        pltpu.VMEM((CH,), jnp.int32),
        pltpu.VMEM((CH, H), jnp.float32),
        pltpu.VMEM((CH, H), jnp.float32),
        pltpu.VMEM((BPT,), jnp.int32),
        pltpu.VMEM((CH, 16), jnp.float32),
        pltpu.SemaphoreType.DMA,
        pltpu.SemaphoreType.DMA,
        pltpu.SemaphoreType.DMA,
    ]
    return pl.kernel(_sc_edge_body, out_type=out, mesh=mesh,
                     scratch_types=scratch, compiler_params=_SC_PARAMS)(
        ss1d, dst1d, g5a, g5b, ci16t, users1d, items1d)


def _tc_head_body(emb, cic, p1a, p1b, b1, p2t, b2, out):
    e = emb[...]
    cc = cic[:, 0:1] * cic[:, 16:17]
    x0 = e[:, 0:H] * e[:, 2 * H:3 * H] * cc
    x1 = e[:, H:2 * H] * e[:, 3 * H:4 * H] * cc
    h = (jnp.dot(x0, p1a[...], preferred_element_type=jnp.float32)
         + jnp.dot(x1, p1b[...], preferred_element_type=jnp.float32)
         + b1[...])
    h = jnp.where(h > 0, h, 0.1 * h)
    out[...] = jnp.dot(h, p2t[...], preferred_element_type=jnp.float32) + b2[...]


def _tc_head(emb, cic, p1a, p1b, b1, p2t, b2):
    bb = 2048
    grid = (B // bb,)
    return pl.pallas_call(
        _tc_head_body,
        grid=grid,
        in_specs=[
            pl.BlockSpec((bb, 4 * H), lambda i: (i, 0)),
            pl.BlockSpec((bb, 32), lambda i: (i, 0)),
            pl.BlockSpec((H, D), lambda i: (0, 0)),
            pl.BlockSpec((H, D), lambda i: (0, 0)),
            pl.BlockSpec((1, D), lambda i: (0, 0)),
            pl.BlockSpec((D, 5), lambda i: (0, 0)),
            pl.BlockSpec((1, 5), lambda i: (0, 0)),
        ],
        out_specs=pl.BlockSpec((bb, 5), lambda i: (i, 0)),
        out_shape=jax.ShapeDtypeStruct((B, 5), jnp.float32),
    )(emb, cic, p1a, p1b, b1, p2t, b2)


def kernel(edge_index, review_feat, score, ci, users, items,
           W_r1, W_r2, S1, S2, S3, feature2, feature3,
           P1_w, P1_b, P2_w, P2_b):
    src = edge_index[0].astype(jnp.int32)
    dst1d = edge_index[1].astype(jnp.int32)
    scr = score.astype(jnp.int32)
    ss1d = jnp.concatenate(
        [src.reshape(NCH, CH), scr.reshape(NCH, CH)], axis=1).reshape(-1)
    users1d = users.astype(jnp.int32)
    items1d = items.astype(jnp.int32)

    f2p = jnp.pad(feature2, ((0, NP_ - N), (0, 0)))
    cip = jnp.pad(ci, ((0, NP_ - N), (0, 0)))

    h0, h1 = _sc_seg_sum(dst1d, review_feat)
    dg0, dg1 = _sc_deg(dst1d)

    h0p = h0.reshape(NP_ // 4, 128)
    h1p = h1.reshape(NP_ // 4, 128)
    deg16 = (dg0.reshape(NP_ // 8, 128)
             + dg1.reshape(NP_ // 8, 128)).reshape(NP_, 16)
    inv = 1.0 / jnp.maximum(deg16[:, :1], 1.0)
    invp = jnp.broadcast_to(inv, (NP_, H)).reshape(NP_ // 4, 128)
    cip32 = jnp.broadcast_to(cip, (NP_, H)).reshape(NP_ // 4, 128)
    f2ap = f2p[:, :H].reshape(NP_ // 4, 128)
    f2bp = f2p[:, H:].reshape(NP_ // 4, 128)
    ci16t = jnp.broadcast_to(cip, (NP_, 16)).reshape(NP_ // 8, 128)

    w2t = W_r2.T
    eye4 = jnp.eye(4, dtype=jnp.float32)
    m0a = jnp.kron(eye4, w2t[:H, :H])
    m1a = jnp.kron(eye4, w2t[H:, :H])
    m0b = jnp.kron(eye4, w2t[:H, H:])
    m1b = jnp.kron(eye4, w2t[H:, H:])
    s2ta = jnp.tile(S2[:, :H], (1, 4))
    s2tb = jnp.tile(S2[:, H:], (1, 4))

    g5a_p, g5b_p = _tc_tables(h0p, h1p, f2ap, f2bp, invp, cip32,
                              m0a, m1a, m0b, m1b, s2ta, s2tb)

    emb, cic = _sc_edge(
        ss1d, dst1d,
        g5a_p.reshape(5 * NP_, H), g5b_p.reshape(5 * NP_, H),
        ci16t.reshape(NP_, 16), users1d, items1d)

    p1t = P1_w.T
    return _tc_head(emb, cic, p1t[:H], p1t[H:], P1_b.reshape(1, D),
                    P2_w.T, P2_b.reshape(1, 5))


# speedup vs baseline: 13.0523x; 1.0011x over previous
import jax
import jax.numpy as jnp
from jax import lax
from jax.experimental import pallas as pl
from jax.experimental.pallas import tpu as pltpu
from jax.experimental.pallas import tpu_sc as plsc

N = 50000
NP_ = 51200
E = 800000
D = 64
H = 32
B = 16384
NS = 16
CH = 256
NCH = E // CH
CPT_BASE = NCH // NS
CPT_EXTRA = NCH % NS
NPT = NP_ // NS
BPT = B // NS

_SC_PARAMS = pltpu.CompilerParams(use_tc_tiling_on_sc=False)


def _tile_chunk_range(s):
    start = s * CPT_BASE + jnp.minimum(s, CPT_EXTRA)
    cnt = CPT_BASE + jnp.where(s < CPT_EXTRA, 1, 0)
    return start, cnt


def _fill_rows(buf, rows, cols, value):
    v = jnp.full((16,), value, jnp.float32)
    for r in range(rows):
        for k in range(cols // 16):
            buf[r, pl.ds(k * 16, 16)] = v


def _zero_shared_slice(zbuf, zrows, shared, s):
    nrep = NPT // zrows
    rem = NPT - nrep * zrows

    def rep(i):
        pltpu.sync_copy(zbuf, shared.at[pl.ds(s * NPT + i * zrows, zrows)])

    pl.loop(0, nrep)(rep)
    if rem:
        pltpu.sync_copy(zbuf.at[pl.ds(0, rem)],
                        shared.at[pl.ds(s * NPT + nrep * zrows, rem)])


def _sc_seg_sum_body(dst1d, rf, h0, h1,
                     acc, dv0, dv1, db0, db1, sem0, sem1, semi):
    c = lax.axis_index("c")
    s = lax.axis_index("s")
    col = c * H

    _fill_rows(db0, CH, H, 0.0)
    _zero_shared_slice(db0, CH, acc, s)
    plsc.subcore_barrier()

    start, cnt = _tile_chunk_range(s)

    def pair(p):
        t0 = start + 2 * p
        t1 = t0 + 1
        l0 = pltpu.async_copy(rf.at[pl.ds(t0 * CH, CH), pl.ds(col, H)],
                              db0, sem0)
        l1 = pltpu.async_copy(rf.at[pl.ds(t1 * CH, CH), pl.ds(col, H)],
                              db1, sem1)
        i0 = pltpu.async_copy(dst1d.at[pl.ds(t0 * CH, CH)], dv0, semi)
        i1 = pltpu.async_copy(dst1d.at[pl.ds(t1 * CH, CH)], dv1, semi)
        l0.wait()
        i0.wait()
        s0 = pltpu.async_copy(db0, acc.at[dv0], sem0, add=True)
        l1.wait()
        i1.wait()
        s1 = pltpu.async_copy(db1, acc.at[dv1], sem1, add=True)
        s0.wait()
        s1.wait()

    pl.loop(0, cnt // 2)(pair)

    @pl.when(cnt % 2 == 1)
    def _():
        t = start + cnt - 1
        pltpu.sync_copy(dst1d.at[pl.ds(t * CH, CH)], dv0)
        pltpu.sync_copy(rf.at[pl.ds(t * CH, CH), pl.ds(col, H)], db0)
        pltpu.sync_copy(db0, acc.at[dv0], add=True)

    plsc.subcore_barrier()
    sl = pl.ds(s * NPT, NPT)

    @pl.when(c == 0)
    def _():
        pltpu.sync_copy(acc.at[sl], h0.at[sl])

    @pl.when(c == 1)
    def _():
        pltpu.sync_copy(acc.at[sl], h1.at[sl])


def _sc_seg_sum(dst1d, rf):
    mesh = plsc.VectorSubcoreMesh(core_axis_name="c", subcore_axis_name="s")
    out = (
        jax.ShapeDtypeStruct((NP_, H), jnp.float32),
        jax.ShapeDtypeStruct((NP_, H), jnp.float32),
    )
    scratch = [
        pltpu.VMEM_SHARED((NP_, H), jnp.float32),
        pltpu.VMEM((CH,), jnp.int32),
        pltpu.VMEM((CH,), jnp.int32),
        pltpu.VMEM((CH, H), jnp.float32),
        pltpu.VMEM((CH, H), jnp.float32),
        pltpu.SemaphoreType.DMA,
        pltpu.SemaphoreType.DMA,
        pltpu.SemaphoreType.DMA,
    ]
    return pl.kernel(_sc_seg_sum_body, out_type=out, mesh=mesh,
                     scratch_types=scratch,
                     compiler_params=_SC_PARAMS)(dst1d, rf)


def _sc_deg_body2(dst1d, dg0, dg1, dacc, dv0, dv1, ones_v, zb, sem0, sem1):
    c = lax.axis_index("c")
    s = lax.axis_index("s")

    _fill_rows(ones_v, CH, 16, 1.0)
    _fill_rows(zb, CH, 16, 0.0)
    _zero_shared_slice(zb, CH, dacc, s)
    plsc.subcore_barrier()

    start, cnt = _tile_chunk_range(s)
    npair = (cnt - c + 3) // 4

    def pair(p):
        i0 = c + 4 * p
        t0 = start + i0
        pltpu.sync_copy(dst1d.at[pl.ds(t0 * CH, CH)], dv0)
        s0 = pltpu.async_copy(ones_v, dacc.at[dv0], sem0, add=True)

        @pl.when(i0 + 2 < cnt)
        def _():
            t1 = start + i0 + 2
            pltpu.sync_copy(dst1d.at[pl.ds(t1 * CH, CH)], dv1)
            s1 = pltpu.async_copy(ones_v, dacc.at[dv1], sem1, add=True)
            s1.wait()

        s0.wait()

    pl.loop(0, npair)(pair)

    plsc.subcore_barrier()
    sl = pl.ds(s * NPT, NPT)

    @pl.when(c == 0)
    def _():
        pltpu.sync_copy(dacc.at[sl], dg0.at[sl])

    @pl.when(c == 1)
    def _():
        pltpu.sync_copy(dacc.at[sl], dg1.at[sl])


def _sc_deg(dst1d):
    mesh = plsc.VectorSubcoreMesh(core_axis_name="c", subcore_axis_name="s")
    out = (
        jax.ShapeDtypeStruct((NP_, 16), jnp.float32),
        jax.ShapeDtypeStruct((NP_, 16), jnp.float32),
    )
    scratch = [
        pltpu.VMEM_SHARED((NP_, 16), jnp.float32),
        pltpu.VMEM((CH,), jnp.int32),
        pltpu.VMEM((CH,), jnp.int32),
        pltpu.VMEM((CH, 16), jnp.float32),
        pltpu.VMEM((CH, 16), jnp.float32),
        pltpu.SemaphoreType.DMA,
        pltpu.SemaphoreType.DMA,
    ]
    return pl.kernel(_sc_deg_body2, out_type=out, mesh=mesh,
                     scratch_types=scratch,
                     compiler_params=_SC_PARAMS)(dst1d)


def _tc_tables_body(h0p, h1p, f2ap, f2bp, invp, cip,
                    m0a, m1a, m0b, m1b, s2ta, s2tb,
                    g5a, g5b):
    iv = invp[...]
    cv = cip[...]
    hr0 = h0p[...] * iv
    hr1 = h1p[...] * iv
    rfa = (jnp.dot(hr0, m0a[...], preferred_element_type=jnp.float32)
           + jnp.dot(hr1, m1a[...], preferred_element_type=jnp.float32))
    rfb = (jnp.dot(hr0, m0b[...], preferred_element_type=jnp.float32)
           + jnp.dot(hr1, m1b[...], preferred_element_type=jnp.float32))
    ga = (f2ap[...] + rfa) * cv
    gb = (f2bp[...] + rfb) * cv
    for k in range(5):
        g5a[k] = ga * s2ta[k, :]
        g5b[k] = gb * s2tb[k, :]


def _tc_tables(h0p, h1p, f2ap, f2bp, invp, cip,
               m0a, m1a, m0b, m1b, s2ta, s2tb):
    bn4 = 256
    grid = (NP_ // (4 * bn4),)
    full = lambda shape: pl.BlockSpec(shape, lambda i: tuple(0 for _ in shape))
    row = pl.BlockSpec((bn4, 128), lambda i: (i, 0))
    return pl.pallas_call(
        _tc_tables_body,
        grid=grid,
        in_specs=[row, row, row, row, row, row,
                  full((128, 128)), full((128, 128)),
                  full((128, 128)), full((128, 128)),
                  full((5, 128)), full((5, 128))],
        out_specs=[
            pl.BlockSpec((5, bn4, 128), lambda i: (0, i, 0)),
            pl.BlockSpec((5, bn4, 128), lambda i: (0, i, 0)),
        ],
        out_shape=[
            jax.ShapeDtypeStruct((5, NP_ // 4, 128), jnp.float32),
            jax.ShapeDtypeStruct((5, NP_ // 4, 128), jnp.float32),
        ],
    )(h0p, h1p, f2ap, f2bp, invp, cip, m0a, m1a, m0b, m1b, s2ta, s2tb)


def _sc_edge_body(ss1d, dst1d, g5a, g5b, ci16t, users1d, items1d,
                  emb, cic,
                  acc, sv0, sv1, dv0, dv1, gx0, gx1, gb0, gb1, uv, cb,
                  sem0, sem1, semi):
    c = lax.axis_index("c")
    s = lax.axis_index("s")

    _fill_rows(gb0, CH, H, 0.0)
    _zero_shared_slice(gb0, CH, acc, s)
    plsc.subcore_barrier()

    start, cnt = _tile_chunk_range(s)

    def build_gidx(sv, gx):
        for k in range(CH // 16):
            sl = pl.ds(k * 16, 16)
            gx[sl] = sv[pl.ds(CH + k * 16, 16)] * NP_ + sv[sl]

    def pair(p, tab):
        t0 = start + 2 * p
        t1 = t0 + 1
        a0 = pltpu.async_copy(ss1d.at[pl.ds(t0 * 2 * CH, 2 * CH)], sv0, semi)
        a1 = pltpu.async_copy(ss1d.at[pl.ds(t1 * 2 * CH, 2 * CH)], sv1, semi)
        b0 = pltpu.async_copy(dst1d.at[pl.ds(t0 * CH, CH)], dv0, semi)
        b1 = pltpu.async_copy(dst1d.at[pl.ds(t1 * CH, CH)], dv1, semi)
        a0.wait()
        build_gidx(sv0, gx0)
        g0 = pltpu.async_copy(tab.at[gx0], gb0, sem0)
        a1.wait()
        build_gidx(sv1, gx1)
        g1 = pltpu.async_copy(tab.at[gx1], gb1, sem1)
        g0.wait()
        b0.wait()
        s0 = pltpu.async_copy(gb0, acc.at[dv0], sem0, add=True)
        g1.wait()
        b1.wait()
        s1 = pltpu.async_copy(gb1, acc.at[dv1], sem1, add=True)
        s0.wait()
        s1.wait()

    def tail(tab):
        @pl.when(cnt % 2 == 1)
        def _():
            t = start + cnt - 1
            pltpu.sync_copy(ss1d.at[pl.ds(t * 2 * CH, 2 * CH)], sv0)
            pltpu.sync_copy(dst1d.at[pl.ds(t * CH, CH)], dv0)
            build_gidx(sv0, gx0)
            pltpu.sync_copy(tab.at[gx0], gb0)
            pltpu.sync_copy(gb0, acc.at[dv0], add=True)

    @pl.when(c == 0)
    def _():
        pl.loop(0, cnt // 2)(lambda p: pair(p, g5a))
        tail(g5a)

    @pl.when(c == 1)
    def _():
        pl.loop(0, cnt // 2)(lambda p: pair(p, g5b))
        tail(g5b)

    plsc.subcore_barrier()

    def bgather(idx1d, col_off, ci_col):
        pltpu.sync_copy(idx1d.at[pl.ds(s * BPT, BPT)], uv)
        ws = []
        for q in range(4):
            gb, sem = (gb0, sem0) if q % 2 == 0 else (gb1, sem1)
            if q >= 2:
                ws[q - 2].wait()
            pltpu.async_copy(acc.at[uv.at[pl.ds(q * CH, CH)]], gb, sem).wait()
            w = pltpu.async_copy(
                gb, emb.at[pl.ds(s * BPT + q * CH, CH), pl.ds(col_off, H)],
                sem)
            ws.append(w)
        ws[2].wait()
        ws[3].wait()
        if ci_col is not None:
            for q in range(4):
                pltpu.sync_copy(ci16t.at[uv.at[pl.ds(q * CH, CH)]], cb)
                pltpu.sync_copy(
                    cb, cic.at[pl.ds(s * BPT + q * CH, CH), pl.ds(ci_col, 16)])

    @pl.when(c == 0)
    def _():
        bgather(users1d, 0, 0)
        bgather(items1d, 2 * H, None)

    @pl.when(c == 1)
    def _():
        bgather(users1d, H, None)
        bgather(items1d, 3 * H, 16)


def _sc_edge(ss1d, dst1d, g5a, g5b, ci16t, users1d, items1d):
    mesh = plsc.VectorSubcoreMesh(core_axis_name="c", subcore_axis_name="s")
    out = (
        jax.ShapeDtypeStruct((B, 4 * H), jnp.float32),
        jax.ShapeDtypeStruct((B, 2 * 16), jnp.float32),
    )
    scratch = [
        pltpu.VMEM_SHARED((NP_, H), jnp.float32),
        pltpu.VMEM((2 * CH,), jnp.int32),
        pltpu.VMEM((2 * CH,), jnp.int32),
        pltpu.VMEM((CH,), jnp.int32),
        pltpu.VMEM((CH,), jnp.int32),
        pltpu.VMEM((CH,), jnp.int32),
        pltpu.VMEM((CH,), jnp.int32),
        pltpu.VMEM((CH, H), jnp.float32),
        pltpu.VMEM((CH, H), jnp.float32),
        pltpu.VMEM((BPT,), jnp.int32),
        pltpu.VMEM((CH, 16), jnp.float32),
        pltpu.SemaphoreType.DMA,
        pltpu.SemaphoreType.DMA,
        pltpu.SemaphoreType.DMA,
    ]
    return pl.kernel(_sc_edge_body, out_type=out, mesh=mesh,
                     scratch_types=scratch, compiler_params=_SC_PARAMS)(
        ss1d, dst1d, g5a, g5b, ci16t, users1d, items1d)


def _tc_head_body(emb, cic, p1a, p1b, b1, p2t, b2, out):
    e = emb[...]
    cc = cic[:, 0:1] * cic[:, 16:17]
    x0 = e[:, 0:H] * e[:, 2 * H:3 * H] * cc
    x1 = e[:, H:2 * H] * e[:, 3 * H:4 * H] * cc
    h = (jnp.dot(x0, p1a[...], preferred_element_type=jnp.float32)
         + jnp.dot(x1, p1b[...], preferred_element_type=jnp.float32)
         + b1[...])
    h = jnp.where(h > 0, h, 0.1 * h)
    out[...] = jnp.dot(h, p2t[...], preferred_element_type=jnp.float32) + b2[...]


def _tc_head(emb, cic, p1a, p1b, b1, p2t, b2):
    bb = 2048
    grid = (B // bb,)
    return pl.pallas_call(
        _tc_head_body,
        grid=grid,
        in_specs=[
            pl.BlockSpec((bb, 4 * H), lambda i: (i, 0)),
            pl.BlockSpec((bb, 32), lambda i: (i, 0)),
            pl.BlockSpec((H, D), lambda i: (0, 0)),
            pl.BlockSpec((H, D), lambda i: (0, 0)),
            pl.BlockSpec((1, D), lambda i: (0, 0)),
            pl.BlockSpec((D, 5), lambda i: (0, 0)),
            pl.BlockSpec((1, 5), lambda i: (0, 0)),
        ],
        out_specs=pl.BlockSpec((bb, 5), lambda i: (i, 0)),
        out_shape=jax.ShapeDtypeStruct((B, 5), jnp.float32),
    )(emb, cic, p1a, p1b, b1, p2t, b2)


def kernel(edge_index, review_feat, score, ci, users, items,
           W_r1, W_r2, S1, S2, S3, feature2, feature3,
           P1_w, P1_b, P2_w, P2_b):
    src = edge_index[0].astype(jnp.int32)
    dst1d = edge_index[1].astype(jnp.int32)
    scr = score.astype(jnp.int32)
    ss1d = jnp.concatenate(
        [src.reshape(NCH, CH), scr.reshape(NCH, CH)], axis=1).reshape(-1)
    users1d = users.astype(jnp.int32)
    items1d = items.astype(jnp.int32)

    f2p = jnp.pad(feature2, ((0, NP_ - N), (0, 0)))
    cip = jnp.pad(ci, ((0, NP_ - N), (0, 0)))

    h0, h1 = _sc_seg_sum(dst1d, review_feat)
    dg0, dg1 = _sc_deg(dst1d)

    h0p = h0.reshape(NP_ // 4, 128)
    h1p = h1.reshape(NP_ // 4, 128)
    deg16 = (dg0.reshape(NP_ // 8, 128)
             + dg1.reshape(NP_ // 8, 128)).reshape(NP_, 16)
    inv = 1.0 / jnp.maximum(deg16[:, :1], 1.0)
    invp = jnp.broadcast_to(inv, (NP_, H)).reshape(NP_ // 4, 128)
    cip32 = jnp.broadcast_to(cip, (NP_, H)).reshape(NP_ // 4, 128)
    f2ap = f2p[:, :H].reshape(NP_ // 4, 128)
    f2bp = f2p[:, H:].reshape(NP_ // 4, 128)
    ci16t = jnp.broadcast_to(cip, (NP_, 16)).reshape(NP_ // 8, 128)

    w2t = W_r2.T
    eye4 = jnp.eye(4, dtype=jnp.float32)
    m0a = jnp.kron(eye4, w2t[:H, :H])
    m1a = jnp.kron(eye4, w2t[H:, :H])
    m0b = jnp.kron(eye4, w2t[:H, H:])
    m1b = jnp.kron(eye4, w2t[H:, H:])
    s2ta = jnp.tile(S2[:, :H], (1, 4))
    s2tb = jnp.tile(S2[:, H:], (1, 4))

    g5a_p, g5b_p = _tc_tables(h0p, h1p, f2ap, f2bp, invp, cip32,
                              m0a, m1a, m0b, m1b, s2ta, s2tb)

    emb, cic = _sc_edge(
        ss1d, dst1d,
        g5a_p.reshape(5 * NP_, H), g5b_p.reshape(5 * NP_, H),
        ci16t.reshape(NP_, 16), users1d, items1d)

    p1t = P1_w.T
    return _tc_head(emb, cic, p1t[:H], p1t[H:], P1_b.reshape(1, D),
                    P2_w.T, P2_b.reshape(1, 5))


# speedup vs baseline: 13.3912x; 1.0260x over previous
import jax
import jax.numpy as jnp
from jax import lax
from jax.experimental import pallas as pl
from jax.experimental.pallas import tpu as pltpu
from jax.experimental.pallas import tpu_sc as plsc

N = 50000
NP_ = 51200
E = 800000
D = 64
H = 32
B = 16384
NS = 16
CH = 256
NCH = E // CH
CPT_BASE = NCH // NS
CPT_EXTRA = NCH % NS
NPT = NP_ // NS
BPT = B // NS

_SC_PARAMS = pltpu.CompilerParams(use_tc_tiling_on_sc=False)


def _tile_chunk_range(s):
    start = s * CPT_BASE + jnp.minimum(s, CPT_EXTRA)
    cnt = CPT_BASE + jnp.where(s < CPT_EXTRA, 1, 0)
    return start, cnt


def _fill_rows(buf, rows, cols, value):
    v = jnp.full((16,), value, jnp.float32)
    for r in range(rows):
        for k in range(cols // 16):
            buf[r, pl.ds(k * 16, 16)] = v


def _zero_shared_slice(zbuf, zrows, shared, s):
    nrep = NPT // zrows
    rem = NPT - nrep * zrows

    def rep(i):
        pltpu.sync_copy(zbuf, shared.at[pl.ds(s * NPT + i * zrows, zrows)])

    pl.loop(0, nrep)(rep)
    if rem:
        pltpu.sync_copy(zbuf.at[pl.ds(0, rem)],
                        shared.at[pl.ds(s * NPT + nrep * zrows, rem)])


def _sc_seg_sum_body(dst1d, rf, h0, h1,
                     acc, dv0, dv1, db0, db1, sem0, sem1, semi):
    c = lax.axis_index("c")
    s = lax.axis_index("s")
    col = c * H

    _fill_rows(db0, CH, H, 0.0)
    _zero_shared_slice(db0, CH, acc, s)
    plsc.subcore_barrier()

    start, cnt = _tile_chunk_range(s)

    def pair(p):
        t0 = start + 2 * p
        t1 = t0 + 1
        l0 = pltpu.async_copy(rf.at[pl.ds(t0 * CH, CH), pl.ds(col, H)],
                              db0, sem0)
        l1 = pltpu.async_copy(rf.at[pl.ds(t1 * CH, CH), pl.ds(col, H)],
                              db1, sem1)
        i0 = pltpu.async_copy(dst1d.at[pl.ds(t0 * CH, CH)], dv0, semi)
        i1 = pltpu.async_copy(dst1d.at[pl.ds(t1 * CH, CH)], dv1, semi)
        l0.wait()
        i0.wait()
        s0 = pltpu.async_copy(db0, acc.at[dv0], sem0, add=True)
        l1.wait()
        i1.wait()
        s1 = pltpu.async_copy(db1, acc.at[dv1], sem1, add=True)
        s0.wait()
        s1.wait()

    pl.loop(0, cnt // 2)(pair)

    @pl.when(cnt % 2 == 1)
    def _():
        t = start + cnt - 1
        pltpu.sync_copy(dst1d.at[pl.ds(t * CH, CH)], dv0)
        pltpu.sync_copy(rf.at[pl.ds(t * CH, CH), pl.ds(col, H)], db0)
        pltpu.sync_copy(db0, acc.at[dv0], add=True)

    plsc.subcore_barrier()
    sl = pl.ds(s * NPT, NPT)

    @pl.when(c == 0)
    def _():
        pltpu.sync_copy(acc.at[sl], h0.at[sl])

    @pl.when(c == 1)
    def _():
        pltpu.sync_copy(acc.at[sl], h1.at[sl])


def _sc_seg_sum(dst1d, rf):
    mesh = plsc.VectorSubcoreMesh(core_axis_name="c", subcore_axis_name="s")
    out = (
        jax.ShapeDtypeStruct((NP_, H), jnp.float32),
        jax.ShapeDtypeStruct((NP_, H), jnp.float32),
    )
    scratch = [
        pltpu.VMEM_SHARED((NP_, H), jnp.float32),
        pltpu.VMEM((CH,), jnp.int32),
        pltpu.VMEM((CH,), jnp.int32),
        pltpu.VMEM((CH, H), jnp.float32),
        pltpu.VMEM((CH, H), jnp.float32),
        pltpu.SemaphoreType.DMA,
        pltpu.SemaphoreType.DMA,
        pltpu.SemaphoreType.DMA,
    ]
    return pl.kernel(_sc_seg_sum_body, out_type=out, mesh=mesh,
                     scratch_types=scratch,
                     compiler_params=_SC_PARAMS)(dst1d, rf)


def _sc_deg_body2(dst1d, dg0, dg1, dacc, dv0, dv1, ones_v, zb, sem0, sem1):
    c = lax.axis_index("c")
    s = lax.axis_index("s")

    _fill_rows(ones_v, CH, 16, 1.0)
    _fill_rows(zb, CH, 16, 0.0)
    _zero_shared_slice(zb, CH, dacc, s)
    plsc.subcore_barrier()

    start, cnt = _tile_chunk_range(s)
    npair = (cnt - c + 3) // 4

    def pair(p):
        i0 = c + 4 * p
        t0 = start + i0
        pltpu.sync_copy(dst1d.at[pl.ds(t0 * CH, CH)], dv0)
        s0 = pltpu.async_copy(ones_v, dacc.at[dv0], sem0, add=True)

        @pl.when(i0 + 2 < cnt)
        def _():
            t1 = start + i0 + 2
            pltpu.sync_copy(dst1d.at[pl.ds(t1 * CH, CH)], dv1)
            s1 = pltpu.async_copy(ones_v, dacc.at[dv1], sem1, add=True)
            s1.wait()

        s0.wait()

    pl.loop(0, npair)(pair)

    plsc.subcore_barrier()
    sl = pl.ds(s * NPT, NPT)

    @pl.when(c == 0)
    def _():
        pltpu.sync_copy(dacc.at[sl], dg0.at[sl])

    @pl.when(c == 1)
    def _():
        pltpu.sync_copy(dacc.at[sl], dg1.at[sl])


def _sc_deg(dst1d):
    mesh = plsc.VectorSubcoreMesh(core_axis_name="c", subcore_axis_name="s")
    out = (
        jax.ShapeDtypeStruct((NP_, 16), jnp.float32),
        jax.ShapeDtypeStruct((NP_, 16), jnp.float32),
    )
    scratch = [
        pltpu.VMEM_SHARED((NP_, 16), jnp.float32),
        pltpu.VMEM((CH,), jnp.int32),
        pltpu.VMEM((CH,), jnp.int32),
        pltpu.VMEM((CH, 16), jnp.float32),
        pltpu.VMEM((CH, 16), jnp.float32),
        pltpu.SemaphoreType.DMA,
        pltpu.SemaphoreType.DMA,
    ]
    return pl.kernel(_sc_deg_body2, out_type=out, mesh=mesh,
                     scratch_types=scratch,
                     compiler_params=_SC_PARAMS)(dst1d)


def _tc_tables_body(h0p, h1p, f2ap, f2bp, invp, cip,
                    m0a, m1a, m0b, m1b, s2ta, s2tb,
                    g5a, g5b):
    iv = invp[...]
    cv = cip[...]
    hr0 = h0p[...] * iv
    hr1 = h1p[...] * iv
    rfa = (jnp.dot(hr0, m0a[...], preferred_element_type=jnp.float32)
           + jnp.dot(hr1, m1a[...], preferred_element_type=jnp.float32))
    rfb = (jnp.dot(hr0, m0b[...], preferred_element_type=jnp.float32)
           + jnp.dot(hr1, m1b[...], preferred_element_type=jnp.float32))
    ga = (f2ap[...] + rfa) * cv
    gb = (f2bp[...] + rfb) * cv
    for k in range(5):
        g5a[k] = ga * s2ta[k, :]
        g5b[k] = gb * s2tb[k, :]


def _tc_tables(h0p, h1p, f2ap, f2bp, invp, cip,
               m0a, m1a, m0b, m1b, s2ta, s2tb):
    bn4 = 256
    grid = (NP_ // (4 * bn4),)
    full = lambda shape: pl.BlockSpec(shape, lambda i: tuple(0 for _ in shape))
    row = pl.BlockSpec((bn4, 128), lambda i: (i, 0))
    return pl.pallas_call(
        _tc_tables_body,
        grid=grid,
        in_specs=[row, row, row, row, row, row,
                  full((128, 128)), full((128, 128)),
                  full((128, 128)), full((128, 128)),
                  full((5, 128)), full((5, 128))],
        out_specs=[
            pl.BlockSpec((5, bn4, 128), lambda i: (0, i, 0)),
            pl.BlockSpec((5, bn4, 128), lambda i: (0, i, 0)),
        ],
        out_shape=[
            jax.ShapeDtypeStruct((5, NP_ // 4, 128), jnp.float32),
            jax.ShapeDtypeStruct((5, NP_ // 4, 128), jnp.float32),
        ],
    )(h0p, h1p, f2ap, f2bp, invp, cip, m0a, m1a, m0b, m1b, s2ta, s2tb)


def _sc_edge_body(ss1d, dst1d, g5a, g5b, ci16t, users1d, items1d,
                  emb, cic,
                  acc, sv0, sv1, sv2, sv3, dv0, dv1, dv2, dv3,
                  gx0, gx1, gx2, gx3, gb0, gb1, uv, cb,
                  sem0, sem1, semi):
    c = lax.axis_index("c")
    s = lax.axis_index("s")

    _fill_rows(gb0, CH, H, 0.0)
    _zero_shared_slice(gb0, CH, acc, s)
    plsc.subcore_barrier()

    start, cnt = _tile_chunk_range(s)

    def build_gidx(sv, gx):
        for k in range(CH // 16):
            sl = pl.ds(k * 16, 16)
            gx[sl] = sv[pl.ds(CH + k * 16, 16)] * NP_ + sv[sl]

    svs = (sv0, sv1, sv2, sv3)
    dvs = (dv0, dv1, dv2, dv3)
    gxs = (gx0, gx1, gx2, gx3)

    def quad(q, tab):
        t0 = start + 4 * q
        aa = [pltpu.async_copy(ss1d.at[pl.ds((t0 + j) * 2 * CH, 2 * CH)],
                               svs[j], semi) for j in range(4)]
        bb = [pltpu.async_copy(dst1d.at[pl.ds((t0 + j) * CH, CH)],
                               dvs[j], semi) for j in range(4)]
        aa[0].wait()
        build_gidx(sv0, gx0)
        g0 = pltpu.async_copy(tab.at[gx0], gb0, sem0)
        aa[1].wait()
        build_gidx(sv1, gx1)
        g1 = pltpu.async_copy(tab.at[gx1], gb1, sem1)
        aa[2].wait()
        build_gidx(sv2, gx2)
        aa[3].wait()
        build_gidx(sv3, gx3)
        g0.wait()
        bb[0].wait()
        s0 = pltpu.async_copy(gb0, acc.at[dv0], sem0, add=True)
        g1.wait()
        bb[1].wait()
        s1 = pltpu.async_copy(gb1, acc.at[dv1], sem1, add=True)
        s0.wait()
        g2 = pltpu.async_copy(tab.at[gx2], gb0, sem0)
        s1.wait()
        g3 = pltpu.async_copy(tab.at[gx3], gb1, sem1)
        g2.wait()
        bb[2].wait()
        s2 = pltpu.async_copy(gb0, acc.at[dv2], sem0, add=True)
        g3.wait()
        bb[3].wait()
        s3 = pltpu.async_copy(gb1, acc.at[dv3], sem1, add=True)
        s2.wait()
        s3.wait()

    def tail(tab):
        def one(i):
            t = start + (cnt // 4) * 4 + i
            pltpu.sync_copy(ss1d.at[pl.ds(t * 2 * CH, 2 * CH)], sv0)
            pltpu.sync_copy(dst1d.at[pl.ds(t * CH, CH)], dv0)
            build_gidx(sv0, gx0)
            pltpu.sync_copy(tab.at[gx0], gb0)
            pltpu.sync_copy(gb0, acc.at[dv0], add=True)
        pl.loop(0, cnt % 4)(one)

    @pl.when(c == 0)
    def _():
        pl.loop(0, cnt // 4)(lambda q: quad(q, g5a))
        tail(g5a)

    @pl.when(c == 1)
    def _():
        pl.loop(0, cnt // 4)(lambda q: quad(q, g5b))
        tail(g5b)

    plsc.subcore_barrier()

    def bgather(idx1d, col_off, ci_col):
        pltpu.sync_copy(idx1d.at[pl.ds(s * BPT, BPT)], uv)
        ws = []
        for q in range(4):
            gb, sem = (gb0, sem0) if q % 2 == 0 else (gb1, sem1)
            if q >= 2:
                ws[q - 2].wait()
            pltpu.async_copy(acc.at[uv.at[pl.ds(q * CH, CH)]], gb, sem).wait()
            w = pltpu.async_copy(
                gb, emb.at[pl.ds(s * BPT + q * CH, CH), pl.ds(col_off, H)],
                sem)
            ws.append(w)
        ws[2].wait()
        ws[3].wait()
        if ci_col is not None:
            for q in range(4):
                pltpu.sync_copy(ci16t.at[uv.at[pl.ds(q * CH, CH)]], cb)
                pltpu.sync_copy(
                    cb, cic.at[pl.ds(s * BPT + q * CH, CH), pl.ds(ci_col, 16)])

    @pl.when(c == 0)
    def _():
        bgather(users1d, 0, 0)
        bgather(items1d, 2 * H, None)

    @pl.when(c == 1)
    def _():
        bgather(users1d, H, None)
        bgather(items1d, 3 * H, 16)


def _sc_edge(ss1d, dst1d, g5a, g5b, ci16t, users1d, items1d):
    mesh = plsc.VectorSubcoreMesh(core_axis_name="c", subcore_axis_name="s")
    out = (
        jax.ShapeDtypeStruct((B, 4 * H), jnp.float32),
        jax.ShapeDtypeStruct((B, 2 * 16), jnp.float32),
    )
    scratch = [
        pltpu.VMEM_SHARED((NP_, H), jnp.float32),
        pltpu.VMEM((2 * CH,), jnp.int32),
        pltpu.VMEM((2 * CH,), jnp.int32),
        pltpu.VMEM((2 * CH,), jnp.int32),
        pltpu.VMEM((2 * CH,), jnp.int32),
        pltpu.VMEM((CH,), jnp.int32),
        pltpu.VMEM((CH,), jnp.int32),
        pltpu.VMEM((CH,), jnp.int32),
        pltpu.VMEM((CH,), jnp.int32),
        pltpu.VMEM((CH,), jnp.int32),
        pltpu.VMEM((CH,), jnp.int32),
        pltpu.VMEM((CH,), jnp.int32),
        pltpu.VMEM((CH,), jnp.int32),
        pltpu.VMEM((CH, H), jnp.float32),
        pltpu.VMEM((CH, H), jnp.float32),
        pltpu.VMEM((BPT,), jnp.int32),
        pltpu.VMEM((CH, 16), jnp.float32),
        pltpu.SemaphoreType.DMA,
        pltpu.SemaphoreType.DMA,
        pltpu.SemaphoreType.DMA,
    ]
    return pl.kernel(_sc_edge_body, out_type=out, mesh=mesh,
                     scratch_types=scratch, compiler_params=_SC_PARAMS)(
        ss1d, dst1d, g5a, g5b, ci16t, users1d, items1d)


def _tc_head_body(emb, cic, p1a, p1b, b1, p2t, b2, out):
    e = emb[...]
    cc = cic[:, 0:1] * cic[:, 16:17]
    x0 = e[:, 0:H] * e[:, 2 * H:3 * H] * cc
    x1 = e[:, H:2 * H] * e[:, 3 * H:4 * H] * cc
    h = (jnp.dot(x0, p1a[...], preferred_element_type=jnp.float32)
         + jnp.dot(x1, p1b[...], preferred_element_type=jnp.float32)
         + b1[...])
    h = jnp.where(h > 0, h, 0.1 * h)
    out[...] = jnp.dot(h, p2t[...], preferred_element_type=jnp.float32) + b2[...]


def _tc_head(emb, cic, p1a, p1b, b1, p2t, b2):
    bb = 2048
    grid = (B // bb,)
    return pl.pallas_call(
        _tc_head_body,
        grid=grid,
        in_specs=[
            pl.BlockSpec((bb, 4 * H), lambda i: (i, 0)),
            pl.BlockSpec((bb, 32), lambda i: (i, 0)),
            pl.BlockSpec((H, D), lambda i: (0, 0)),
            pl.BlockSpec((H, D), lambda i: (0, 0)),
            pl.BlockSpec((1, D), lambda i: (0, 0)),
            pl.BlockSpec((D, 5), lambda i: (0, 0)),
            pl.BlockSpec((1, 5), lambda i: (0, 0)),
        ],
        out_specs=pl.BlockSpec((bb, 5), lambda i: (i, 0)),
        out_shape=jax.ShapeDtypeStruct((B, 5), jnp.float32),
    )(emb, cic, p1a, p1b, b1, p2t, b2)


def kernel(edge_index, review_feat, score, ci, users, items,
           W_r1, W_r2, S1, S2, S3, feature2, feature3,
           P1_w, P1_b, P2_w, P2_b):
    src = edge_index[0].astype(jnp.int32)
    dst1d = edge_index[1].astype(jnp.int32)
    scr = score.astype(jnp.int32)
    ss1d = jnp.concatenate(
        [src.reshape(NCH, CH), scr.reshape(NCH, CH)], axis=1).reshape(-1)
    users1d = users.astype(jnp.int32)
    items1d = items.astype(jnp.int32)

    f2p = jnp.pad(feature2, ((0, NP_ - N), (0, 0)))
    cip = jnp.pad(ci, ((0, NP_ - N), (0, 0)))

    h0, h1 = _sc_seg_sum(dst1d, review_feat)
    dg0, dg1 = _sc_deg(dst1d)

    h0p = h0.reshape(NP_ // 4, 128)
    h1p = h1.reshape(NP_ // 4, 128)
    deg16 = (dg0.reshape(NP_ // 8, 128)
             + dg1.reshape(NP_ // 8, 128)).reshape(NP_, 16)
    inv = 1.0 / jnp.maximum(deg16[:, :1], 1.0)
    invp = jnp.broadcast_to(inv, (NP_, H)).reshape(NP_ // 4, 128)
    cip32 = jnp.broadcast_to(cip, (NP_, H)).reshape(NP_ // 4, 128)
    f2ap = f2p[:, :H].reshape(NP_ // 4, 128)
    f2bp = f2p[:, H:].reshape(NP_ // 4, 128)
    ci16t = jnp.broadcast_to(cip, (NP_, 16)).reshape(NP_ // 8, 128)

    w2t = W_r2.T
    eye4 = jnp.eye(4, dtype=jnp.float32)
    m0a = jnp.kron(eye4, w2t[:H, :H])
    m1a = jnp.kron(eye4, w2t[H:, :H])
    m0b = jnp.kron(eye4, w2t[:H, H:])
    m1b = jnp.kron(eye4, w2t[H:, H:])
    s2ta = jnp.tile(S2[:, :H], (1, 4))
    s2tb = jnp.tile(S2[:, H:], (1, 4))

    g5a_p, g5b_p = _tc_tables(h0p, h1p, f2ap, f2bp, invp, cip32,
                              m0a, m1a, m0b, m1b, s2ta, s2tb)

    emb, cic = _sc_edge(
        ss1d, dst1d,
        g5a_p.reshape(5 * NP_, H), g5b_p.reshape(5 * NP_, H),
        ci16t.reshape(NP_, 16), users1d, items1d)

    p1t = P1_w.T
    return _tc_head(emb, cic, p1t[:H], p1t[H:], P1_b.reshape(1, D),
                    P2_w.T, P2_b.reshape(1, 5))
